# Initial kernel scaffold; baseline (speedup 1.0000x reference)
import jax
import jax.numpy as jnp
from jax.experimental import pallas as pl


def kernel(xyz, atom_xyz, atom_features, batch, atom_batch, params):
    raise NotImplementedError("write your pallas kernel here")


# speedup vs baseline: 8.4426x; 8.4426x over previous
import functools

import jax
import jax.numpy as jnp
from jax import lax
from jax.experimental import pallas as pl
from jax.experimental.pallas import tpu as pltpu
from jax.experimental.pallas import tpu_sc as plsc

D = 16
K = 16
N_LAYERS = 3
H = 2 * D + 1
MASKVAL = 1e10
DONEVAL = 3e10


def _knn_body(qs_ref, ks_ref, q2_ref, k2_ref, qb_ref, kb_ref,
              idx_ref, dist_ref, *, k_sel, nk_pad):
    qs = qs_ref[...]
    ks = ks_ref[...]
    cross = lax.dot_general(qs, ks, (((0,), (0,)), ((), ())),
                            preferred_element_type=jnp.float32)
    d = (q2_ref[...] + k2_ref[...]) - 2.0 * cross
    mask = qb_ref[...] != kb_ref[...]
    d = jnp.where(mask, MASKVAL, d)
    iota = lax.broadcasted_iota(jnp.int32, d.shape, 1)
    for p in range(k_sel):
        m = jnp.min(d, axis=1, keepdims=True)
        jj = jnp.min(jnp.where(d == m, iota, nk_pad), axis=1,
                     keepdims=True)
        dist_ref[:, p:p + 1] = m
        idx_ref[:, p:p + 1] = jj
        d = jnp.where(iota == jj, DONEVAL, d)


def _knn(q_xyz, k_xyz, q_batch, k_batch, k_sel, qb_rows=128):
    nq, nk = q_xyz.shape[0], k_xyz.shape[0]
    nq_pad = ((nq + qb_rows - 1) // qb_rows) * qb_rows
    nk_pad = ((nk + 127) // 128) * 128

    def stack(c, n_pad):
        r = jnp.concatenate([c.T, jnp.zeros((5, c.shape[0]), jnp.float32)], 0)
        return jnp.pad(r, ((0, 0), (0, n_pad - c.shape[0])))

    qs = stack(q_xyz, nq_pad)
    ks = stack(k_xyz, nk_pad)
    q2 = jnp.pad(jnp.sum(q_xyz * q_xyz, axis=1), (0, nq_pad - nq))[:, None]
    k2 = jnp.pad(jnp.sum(k_xyz * k_xyz, axis=1), (0, nk_pad - nk))[None, :]
    qb = jnp.pad(q_batch.astype(jnp.int32), (0, nq_pad - nq),
                 constant_values=-1)[:, None]
    kb = jnp.pad(k_batch.astype(jnp.int32), (0, nk_pad - nk),
                 constant_values=-2)[None, :]

    grid = (nq_pad // qb_rows,)
    idx, dist = pl.pallas_call(
        functools.partial(_knn_body, k_sel=k_sel, nk_pad=nk_pad),
        grid=grid,
        in_specs=[
            pl.BlockSpec((8, qb_rows), lambda i: (0, i)),
            pl.BlockSpec((8, nk_pad), lambda i: (0, 0)),
            pl.BlockSpec((qb_rows, 1), lambda i: (i, 0)),
            pl.BlockSpec((1, nk_pad), lambda i: (0, 0)),
            pl.BlockSpec((qb_rows, 1), lambda i: (i, 0)),
            pl.BlockSpec((1, nk_pad), lambda i: (0, 0)),
        ],
        out_specs=[
            pl.BlockSpec((qb_rows, k_sel), lambda i: (i, 0)),
            pl.BlockSpec((qb_rows, k_sel), lambda i: (i, 0)),
        ],
        out_shape=[
            jax.ShapeDtypeStruct((nq_pad, k_sel), jnp.int32),
            jax.ShapeDtypeStruct((nq_pad, k_sel), jnp.float32),
        ],
    )(qs, ks, q2, k2, qb, kb)
    return idx, dist


def _sc_gather(table, idx):
    B = idx.shape[0]
    NW = 32
    bpw = B // NW
    CH = 128
    n_chunks = bpw // CH
    mesh = plsc.VectorSubcoreMesh(core_axis_name="c", subcore_axis_name="s")

    @functools.partial(
        pl.kernel, mesh=mesh,
        compiler_params=pltpu.CompilerParams(use_tc_tiling_on_sc=False),
        out_type=jax.ShapeDtypeStruct((B, D), jnp.float32),
        scratch_types=[
            pltpu.VMEM((bpw,), jnp.int32),
            pltpu.VMEM((bpw, D), jnp.float32),
            pltpu.SemaphoreType.DMA,
        ],
    )
    def gk(table_hbm, idx_hbm, out_hbm, idx_v, rows_v, sem):
        wid = lax.axis_index("s") * 2 + lax.axis_index("c")
        base = wid * bpw
        pltpu.sync_copy(idx_hbm.at[pl.ds(base, bpw)], idx_v)

        def chunk(c, carry):
            off = c * CH
            pltpu.async_copy(table_hbm.at[idx_v.at[pl.ds(off, CH)]],
                             rows_v.at[pl.ds(off, CH)], sem).wait()
            return carry

        lax.fori_loop(0, n_chunks, chunk, 0)
        pltpu.sync_copy(rows_v, out_hbm.at[pl.ds(base, bpw)])

    return gk(table, idx)


def _layer_body(x_ref, g_ref, dd_ref, a1_ref, bm_ref, c1_ref, b1_ref,
                w2t_ref, b2_ref, gnw_ref, gnb_ref, m8_ref, out_ref):
    x = x_ref[...]
    xi = jnp.dot(x, a1_ref[...],
                 preferred_element_type=jnp.float32)
    c1 = c1_ref[...]
    b1 = b1_ref[...]
    w2t = w2t_ref[...]
    acc = jnp.zeros_like(x)
    for k in range(K):
        gk = g_ref[k]
        h = xi + jnp.dot(gk, bm_ref[...],
                         preferred_element_type=jnp.float32)
        h = h + dd_ref[:, k:k + 1] * c1 + b1
        h = jnp.maximum(h, 0.2 * h)
        acc = acc + jnp.dot(h, w2t, preferred_element_type=jnp.float32)
    msum = acc + jnp.float32(K) * b2_ref[...]
    mu = jnp.dot(msum, m8_ref[...], preferred_element_type=jnp.float32)
    cent = msum - mu
    var = jnp.dot(cent * cent, m8_ref[...],
                  preferred_element_type=jnp.float32)
    y = (cent / jnp.sqrt(var + 1e-5)) * gnw_ref[...] + gnb_ref[...]
    out_ref[...] = x + jnp.maximum(y, 0.2 * y)


def _mp_layer(x, g3, dd, w1, b1, w2, b2, gnw, gnb, nb_rows=512):
    n = x.shape[0]
    a1 = w1[:, :D].T
    bm = w1[:, D:2 * D].T
    c1 = w1[:, 2 * D][None, :]
    m8 = jnp.zeros((D, D), jnp.float32)
    m8 = m8.at[:8, :8].set(1.0 / 8).at[8:, 8:].set(1.0 / 8)

    grid = (n // nb_rows,)
    return pl.pallas_call(
        _layer_body,
        grid=grid,
        in_specs=[
            pl.BlockSpec((nb_rows, D), lambda i: (i, 0)),
            pl.BlockSpec((K, nb_rows, D), lambda i: (0, i, 0)),
            pl.BlockSpec((nb_rows, K), lambda i: (i, 0)),
            pl.BlockSpec((D, H), lambda i: (0, 0)),
            pl.BlockSpec((D, H), lambda i: (0, 0)),
            pl.BlockSpec((1, H), lambda i: (0, 0)),
            pl.BlockSpec((1, H), lambda i: (0, 0)),
            pl.BlockSpec((H, D), lambda i: (0, 0)),
            pl.BlockSpec((1, D), lambda i: (0, 0)),
            pl.BlockSpec((1, D), lambda i: (0, 0)),
            pl.BlockSpec((1, D), lambda i: (0, 0)),
            pl.BlockSpec((D, D), lambda i: (0, 0)),
        ],
        out_specs=pl.BlockSpec((nb_rows, D), lambda i: (i, 0)),
        out_shape=jax.ShapeDtypeStruct((n, D), jnp.float32),
    )(x, g3, dd, a1, bm, c1, b1[None, :], w2.T, b2[None, :],
      gnw[None, :], gnb[None, :], m8)


def _tf_body(x_ref, w1t_ref, b1_ref, w2t_ref, b2_ref, out_ref):
    h = jnp.dot(x_ref[...], w1t_ref[...],
                preferred_element_type=jnp.float32) + b1_ref[...]
    h = jnp.maximum(h, 0.2 * h)
    out_ref[...] = jnp.dot(h, w2t_ref[...],
                           preferred_element_type=jnp.float32) + b2_ref[...]


def _tf(x, w1, b1, w2, b2):
    n = x.shape[0]
    return pl.pallas_call(
        _tf_body,
        grid=(1,),
        in_specs=[
            pl.BlockSpec((n, D), lambda i: (0, 0)),
            pl.BlockSpec((D, D), lambda i: (0, 0)),
            pl.BlockSpec((1, D), lambda i: (0, 0)),
            pl.BlockSpec((D, D), lambda i: (0, 0)),
            pl.BlockSpec((1, D), lambda i: (0, 0)),
        ],
        out_specs=pl.BlockSpec((n, D), lambda i: (0, 0)),
        out_shape=jax.ShapeDtypeStruct((n, D), jnp.float32),
    )(x, w1.T, b1[None, :], w2.T, b2[None, :])


def kernel(xyz, atom_xyz, atom_features, batch, atom_batch, params):
    p = params
    n_surf, n_atom = xyz.shape[0], atom_xyz.shape[0]
    na_pad = ((n_atom + 511) // 512) * 512
    ns_pad = ((n_surf + 511) // 512) * 512

    af_pad = jnp.pad(atom_features, ((0, na_pad - n_atom), (0, 0)))
    out = _tf(af_pad, p['tf_w1'], p['tf_b1'], p['tf_w2'], p['tf_b2'])

    idx, dist = _knn(atom_xyz, atom_xyz, atom_batch, atom_batch, K + 1)
    idx = idx[:n_atom, 1:]
    dd = jnp.pad(dist[:n_atom, 1:], ((0, na_pad - n_atom), (0, 0)))
    idx_flat = jnp.pad(idx.T.astype(jnp.int32),
                       ((0, 0), (0, na_pad - n_atom))).reshape(-1)
    for i in range(N_LAYERS):
        g = _sc_gather(out, idx_flat).reshape(K, na_pad, D)
        out = _mp_layer(out, g, dd,
                        p['aa_w1_' + str(i)], p['aa_b1_' + str(i)],
                        p['aa_w2_' + str(i)], p['aa_b2_' + str(i)],
                        p['aa_gn_w_' + str(i)], p['aa_gn_b_' + str(i)])

    idx2, dist2 = _knn(xyz, atom_xyz, batch, atom_batch, K)
    dd2 = jnp.pad(dist2[:n_surf, :], ((0, ns_pad - n_surf), (0, 0)))
    idx2_flat = jnp.pad(idx2[:n_surf].T.astype(jnp.int32),
                        ((0, 0), (0, ns_pad - n_surf))).reshape(-1)
    g2 = _sc_gather(out, idx2_flat).reshape(K, ns_pad, D)
    pe = jnp.ones((ns_pad, D), jnp.float32)
    for i in range(N_LAYERS):
        pe = _mp_layer(pe, g2, dd2,
                       p['ae_w1_' + str(i)], p['ae_b1_' + str(i)],
                       p['ae_w2_' + str(i)], p['ae_b2_' + str(i)],
                       p['ae_gn_w_' + str(i)], p['ae_gn_b_' + str(i)])
    return pe[:n_surf]


# speedup vs baseline: 17.3310x; 2.0528x over previous
import functools

import jax
import jax.numpy as jnp
from jax import lax
from jax.experimental import pallas as pl
from jax.experimental.pallas import tpu as pltpu
from jax.experimental.pallas import tpu_sc as plsc

D = 16
K = 16
N_LAYERS = 3
H = 2 * D + 1
MASKVAL = 1e10
DONEVAL = 3e10


_TW = 512
_IBIG = 1 << 30


def _knn_body(lohi_ref, qs_ref, ks_ref, q2_ref, k2_ref, qb_ref, kb_ref,
              idx_ref, dist_ref, *, k_sel):
    i = pl.program_id(0)
    lo = lohi_ref[0, i]
    hi = lohi_ref[1, i]
    qs = qs_ref[...]
    q2 = q2_ref[...]
    qb = qb_ref[...]

    cds = tuple(jnp.full(q2.shape, MASKVAL, jnp.float32)
                for _ in range(k_sel))
    cis = tuple(jnp.full(q2.shape, p, jnp.int32) for p in range(k_sel))

    def tile_step(t, carry):
        cds, cis = carry
        off = pl.multiple_of(t * _TW, _TW)
        kst = ks_ref[pl.ds(off, _TW), :]
        k2t = k2_ref[pl.ds(off, _TW), :]
        kbt = kb_ref[pl.ds(off, _TW), :]
        cross = lax.dot_general(kst, qs, (((1,), (0,)), ((), ())),
                                preferred_element_type=jnp.float32)
        dt = (q2 + k2t) - 2.0 * cross
        dt = jnp.where(kbt != qb, MASKVAL, dt)
        it = lax.broadcasted_iota(jnp.int32, dt.shape, 0) + off
        ad = jnp.concatenate(list(cds) + [dt], axis=0)
        ai = jnp.concatenate(list(cis) + [it], axis=0)
        ncd, nci = [], []
        for _ in range(k_sel):
            m = jnp.min(ad, axis=0, keepdims=True)
            ii = jnp.min(jnp.where(ad == m, ai, _IBIG), axis=0,
                         keepdims=True)
            ncd.append(m)
            nci.append(ii)
            ad = jnp.where((ad == m) & (ai == ii), DONEVAL, ad)
        return tuple(ncd), tuple(nci)

    cds, cis = lax.fori_loop(lo, hi, tile_step, (cds, cis))
    for p in range(k_sel):
        dist_ref[p:p + 1, :] = cds[p]
        idx_ref[p:p + 1, :] = cis[p]


def _knn(q_xyz, k_xyz, q_batch, k_batch, k_sel, qb_rows=128):
    nq, nk = q_xyz.shape[0], k_xyz.shape[0]
    nq_pad = ((nq + qb_rows - 1) // qb_rows) * qb_rows
    nk_pad = ((nk + _TW - 1) // _TW) * _TW
    nblk = nq_pad // qb_rows
    kp = ((k_sel + 7) // 8) * 8

    qs = jnp.pad(q_xyz.T, ((0, 5), (0, nq_pad - nq)))
    ks = jnp.pad(k_xyz, ((0, nk_pad - nk), (0, 5)))
    q2 = jnp.pad(jnp.sum(q_xyz * q_xyz, axis=1), (0, nq_pad - nq))[None, :]
    k2 = jnp.pad(jnp.sum(k_xyz * k_xyz, axis=1), (0, nk_pad - nk))[:, None]
    qb = jnp.pad(q_batch.astype(jnp.int32), (0, nq_pad - nq),
                 constant_values=1 << 20)[None, :]
    kb = jnp.pad(k_batch.astype(jnp.int32), (0, nk_pad - nk),
                 constant_values=-2)[:, None]

    blo = qb[0, ::qb_rows]
    bhi = qb[0, qb_rows - 1::qb_rows]
    kstart = jnp.searchsorted(k_batch, blo).astype(jnp.int32)
    kend = jnp.searchsorted(k_batch, bhi, side='right').astype(jnp.int32)
    lohi = jnp.stack([kstart // _TW, (kend + _TW - 1) // _TW], axis=0)

    idx, dist = pl.pallas_call(
        functools.partial(_knn_body, k_sel=k_sel),
        grid=(nblk,),
        in_specs=[
            pl.BlockSpec(memory_space=pltpu.SMEM),
            pl.BlockSpec((8, qb_rows), lambda i: (0, i)),
            pl.BlockSpec((nk_pad, 8), lambda i: (0, 0)),
            pl.BlockSpec((1, qb_rows), lambda i: (0, i)),
            pl.BlockSpec((nk_pad, 1), lambda i: (0, 0)),
            pl.BlockSpec((1, qb_rows), lambda i: (0, i)),
            pl.BlockSpec((nk_pad, 1), lambda i: (0, 0)),
        ],
        out_specs=[
            pl.BlockSpec((kp, qb_rows), lambda i: (0, i)),
            pl.BlockSpec((kp, qb_rows), lambda i: (0, i)),
        ],
        out_shape=[
            jax.ShapeDtypeStruct((kp, nq_pad), jnp.int32),
            jax.ShapeDtypeStruct((kp, nq_pad), jnp.float32),
        ],
    )(lohi, qs, ks, q2, k2, qb, kb)
    return idx[:k_sel, :nq].T, dist[:k_sel, :nq].T


def _sc_gather(table, idx):
    B = idx.shape[0]
    NW = 32
    bpw = B // NW
    CH = 128
    n_chunks = bpw // CH
    mesh = plsc.VectorSubcoreMesh(core_axis_name="c", subcore_axis_name="s")

    @functools.partial(
        pl.kernel, mesh=mesh,
        compiler_params=pltpu.CompilerParams(use_tc_tiling_on_sc=False),
        out_type=jax.ShapeDtypeStruct((B, D), jnp.float32),
        scratch_types=[
            pltpu.VMEM((bpw,), jnp.int32),
            pltpu.VMEM((bpw, D), jnp.float32),
            pltpu.SemaphoreType.DMA,
        ],
    )
    def gk(table_hbm, idx_hbm, out_hbm, idx_v, rows_v, sem):
        wid = lax.axis_index("s") * 2 + lax.axis_index("c")
        base = wid * bpw
        pltpu.sync_copy(idx_hbm.at[pl.ds(base, bpw)], idx_v)

        def chunk(c, carry):
            off = c * CH
            pltpu.async_copy(table_hbm.at[idx_v.at[pl.ds(off, CH)]],
                             rows_v.at[pl.ds(off, CH)], sem).wait()
            return carry

        lax.fori_loop(0, n_chunks, chunk, 0)
        pltpu.sync_copy(rows_v, out_hbm.at[pl.ds(base, bpw)])

    return gk(table, idx)


def _layer_body(x_ref, g_ref, dd_ref, a1_ref, bm_ref, c1_ref, b1_ref,
                w2t_ref, b2_ref, gnw_ref, gnb_ref, m8_ref, out_ref):
    x = x_ref[...]
    xi = jnp.dot(x, a1_ref[...],
                 preferred_element_type=jnp.float32)
    c1 = c1_ref[...]
    b1 = b1_ref[...]
    w2t = w2t_ref[...]
    acc = jnp.zeros_like(x)
    for k in range(K):
        gk = g_ref[k]
        h = xi + jnp.dot(gk, bm_ref[...],
                         preferred_element_type=jnp.float32)
        h = h + dd_ref[:, k:k + 1] * c1 + b1
        h = jnp.maximum(h, 0.2 * h)
        acc = acc + jnp.dot(h, w2t, preferred_element_type=jnp.float32)
    msum = acc + jnp.float32(K) * b2_ref[...]
    mu = jnp.dot(msum, m8_ref[...], preferred_element_type=jnp.float32)
    cent = msum - mu
    var = jnp.dot(cent * cent, m8_ref[...],
                  preferred_element_type=jnp.float32)
    y = (cent / jnp.sqrt(var + 1e-5)) * gnw_ref[...] + gnb_ref[...]
    out_ref[...] = x + jnp.maximum(y, 0.2 * y)


def _mp_layer(x, g3, dd, w1, b1, w2, b2, gnw, gnb, nb_rows=512):
    n = x.shape[0]
    a1 = w1[:, :D].T
    bm = w1[:, D:2 * D].T
    c1 = w1[:, 2 * D][None, :]
    m8 = jnp.zeros((D, D), jnp.float32)
    m8 = m8.at[:8, :8].set(1.0 / 8).at[8:, 8:].set(1.0 / 8)

    grid = (n // nb_rows,)
    return pl.pallas_call(
        _layer_body,
        grid=grid,
        in_specs=[
            pl.BlockSpec((nb_rows, D), lambda i: (i, 0)),
            pl.BlockSpec((K, nb_rows, D), lambda i: (0, i, 0)),
            pl.BlockSpec((nb_rows, K), lambda i: (i, 0)),
            pl.BlockSpec((D, H), lambda i: (0, 0)),
            pl.BlockSpec((D, H), lambda i: (0, 0)),
            pl.BlockSpec((1, H), lambda i: (0, 0)),
            pl.BlockSpec((1, H), lambda i: (0, 0)),
            pl.BlockSpec((H, D), lambda i: (0, 0)),
            pl.BlockSpec((1, D), lambda i: (0, 0)),
            pl.BlockSpec((1, D), lambda i: (0, 0)),
            pl.BlockSpec((1, D), lambda i: (0, 0)),
            pl.BlockSpec((D, D), lambda i: (0, 0)),
        ],
        out_specs=pl.BlockSpec((nb_rows, D), lambda i: (i, 0)),
        out_shape=jax.ShapeDtypeStruct((n, D), jnp.float32),
    )(x, g3, dd, a1, bm, c1, b1[None, :], w2.T, b2[None, :],
      gnw[None, :], gnb[None, :], m8)


def _tf_body(x_ref, w1t_ref, b1_ref, w2t_ref, b2_ref, out_ref):
    h = jnp.dot(x_ref[...], w1t_ref[...],
                preferred_element_type=jnp.float32) + b1_ref[...]
    h = jnp.maximum(h, 0.2 * h)
    out_ref[...] = jnp.dot(h, w2t_ref[...],
                           preferred_element_type=jnp.float32) + b2_ref[...]


def _tf(x, w1, b1, w2, b2):
    n = x.shape[0]
    return pl.pallas_call(
        _tf_body,
        grid=(1,),
        in_specs=[
            pl.BlockSpec((n, D), lambda i: (0, 0)),
            pl.BlockSpec((D, D), lambda i: (0, 0)),
            pl.BlockSpec((1, D), lambda i: (0, 0)),
            pl.BlockSpec((D, D), lambda i: (0, 0)),
            pl.BlockSpec((1, D), lambda i: (0, 0)),
        ],
        out_specs=pl.BlockSpec((n, D), lambda i: (0, 0)),
        out_shape=jax.ShapeDtypeStruct((n, D), jnp.float32),
    )(x, w1.T, b1[None, :], w2.T, b2[None, :])


def kernel(xyz, atom_xyz, atom_features, batch, atom_batch, params):
    p = params
    n_surf, n_atom = xyz.shape[0], atom_xyz.shape[0]
    na_pad = ((n_atom + 511) // 512) * 512
    ns_pad = ((n_surf + 511) // 512) * 512

    af_pad = jnp.pad(atom_features, ((0, na_pad - n_atom), (0, 0)))
    out = _tf(af_pad, p['tf_w1'], p['tf_b1'], p['tf_w2'], p['tf_b2'])

    idx, dist = _knn(atom_xyz, atom_xyz, atom_batch, atom_batch, K + 1)
    idx = idx[:, 1:]
    dd = jnp.pad(dist[:, 1:], ((0, na_pad - n_atom), (0, 0)))
    idx_flat = jnp.pad(idx.T.astype(jnp.int32),
                       ((0, 0), (0, na_pad - n_atom))).reshape(-1)
    for i in range(N_LAYERS):
        g = _sc_gather(out, idx_flat).reshape(K, na_pad, D)
        out = _mp_layer(out, g, dd,
                        p['aa_w1_' + str(i)], p['aa_b1_' + str(i)],
                        p['aa_w2_' + str(i)], p['aa_b2_' + str(i)],
                        p['aa_gn_w_' + str(i)], p['aa_gn_b_' + str(i)])

    idx2, dist2 = _knn(xyz, atom_xyz, batch, atom_batch, K)
    dd2 = jnp.pad(dist2, ((0, ns_pad - n_surf), (0, 0)))
    idx2_flat = jnp.pad(idx2.T.astype(jnp.int32),
                        ((0, 0), (0, ns_pad - n_surf))).reshape(-1)
    g2 = _sc_gather(out, idx2_flat).reshape(K, ns_pad, D)
    pe = jnp.ones((ns_pad, D), jnp.float32)
    for i in range(N_LAYERS):
        pe = _mp_layer(pe, g2, dd2,
                       p['ae_w1_' + str(i)], p['ae_b1_' + str(i)],
                       p['ae_w2_' + str(i)], p['ae_b2_' + str(i)],
                       p['ae_gn_w_' + str(i)], p['ae_gn_b_' + str(i)])
    return pe[:n_surf]


# speedup vs baseline: 18.7063x; 1.0794x over previous
import functools

import jax
import jax.numpy as jnp
from jax import lax
from jax.experimental import pallas as pl
from jax.experimental.pallas import tpu as pltpu
from jax.experimental.pallas import tpu_sc as plsc

D = 16
K = 16
N_LAYERS = 3
H = 2 * D + 1
MASKVAL = 1e10
DONEVAL = 3e10


_TW = 512
_IBIG = 1 << 30


def _knn_body(lohi_ref, qs_ref, ks_ref, q2_ref, k2_ref, qb_ref, kb_ref,
              idx_ref, dist_ref, *, k_sel):
    i = pl.program_id(0)
    lo = lohi_ref[0, i]
    hi = lohi_ref[1, i]
    qs = qs_ref[...]
    q2 = q2_ref[...]
    qb = qb_ref[...]

    cds = tuple(jnp.full(q2.shape, MASKVAL, jnp.float32)
                for _ in range(k_sel))
    cis = tuple(jnp.full(q2.shape, p, jnp.int32) for p in range(k_sel))

    def tile_step(t, carry):
        cds, cis = carry
        off = pl.multiple_of(t * _TW, _TW)
        kst = ks_ref[pl.ds(off, _TW), :]
        k2t = k2_ref[pl.ds(off, _TW), :]
        kbt = kb_ref[pl.ds(off, _TW), :]
        cross = lax.dot_general(kst, qs, (((1,), (0,)), ((), ())),
                                preferred_element_type=jnp.float32)
        dt = (q2 + k2t) - 2.0 * cross
        dt = jnp.where(kbt != qb, MASKVAL, dt)
        it = lax.broadcasted_iota(jnp.int32, dt.shape, 0) + off
        ad = jnp.concatenate(list(cds) + [dt], axis=0)
        ai = jnp.concatenate(list(cis) + [it], axis=0)
        ncd, nci = [], []
        for _ in range(k_sel):
            m = jnp.min(ad, axis=0, keepdims=True)
            ii = jnp.min(jnp.where(ad == m, ai, _IBIG), axis=0,
                         keepdims=True)
            ncd.append(m)
            nci.append(ii)
            ad = jnp.where((ad == m) & (ai == ii), DONEVAL, ad)
        return tuple(ncd), tuple(nci)

    cds, cis = lax.fori_loop(lo, hi, tile_step, (cds, cis))
    for p in range(k_sel):
        dist_ref[p:p + 1, :] = cds[p]
        idx_ref[p:p + 1, :] = cis[p]


def _knn(q_xyz, k_xyz, q_batch, k_batch, k_sel, qb_rows=128):
    nq, nk = q_xyz.shape[0], k_xyz.shape[0]
    nq_pad = ((nq + qb_rows - 1) // qb_rows) * qb_rows
    nk_pad = ((nk + _TW - 1) // _TW) * _TW
    nblk = nq_pad // qb_rows
    kp = ((k_sel + 7) // 8) * 8

    qs = jnp.pad(q_xyz.T, ((0, 5), (0, nq_pad - nq)))
    ks = jnp.pad(k_xyz, ((0, nk_pad - nk), (0, 5)))
    q2 = jnp.pad(jnp.sum(q_xyz * q_xyz, axis=1), (0, nq_pad - nq))[None, :]
    k2 = jnp.pad(jnp.sum(k_xyz * k_xyz, axis=1), (0, nk_pad - nk))[:, None]
    qb = jnp.pad(q_batch.astype(jnp.int32), (0, nq_pad - nq),
                 constant_values=1 << 20)[None, :]
    kb = jnp.pad(k_batch.astype(jnp.int32), (0, nk_pad - nk),
                 constant_values=-2)[:, None]

    blo = qb[0, ::qb_rows]
    bhi = qb[0, qb_rows - 1::qb_rows]
    kstart = jnp.searchsorted(k_batch, blo).astype(jnp.int32)
    kend = jnp.searchsorted(k_batch, bhi, side='right').astype(jnp.int32)
    lohi = jnp.stack([kstart // _TW, (kend + _TW - 1) // _TW], axis=0)

    idx, dist = pl.pallas_call(
        functools.partial(_knn_body, k_sel=k_sel),
        grid=(nblk,),
        in_specs=[
            pl.BlockSpec(memory_space=pltpu.SMEM),
            pl.BlockSpec((8, qb_rows), lambda i: (0, i)),
            pl.BlockSpec((nk_pad, 8), lambda i: (0, 0)),
            pl.BlockSpec((1, qb_rows), lambda i: (0, i)),
            pl.BlockSpec((nk_pad, 1), lambda i: (0, 0)),
            pl.BlockSpec((1, qb_rows), lambda i: (0, i)),
            pl.BlockSpec((nk_pad, 1), lambda i: (0, 0)),
        ],
        out_specs=[
            pl.BlockSpec((kp, qb_rows), lambda i: (0, i)),
            pl.BlockSpec((kp, qb_rows), lambda i: (0, i)),
        ],
        out_shape=[
            jax.ShapeDtypeStruct((kp, nq_pad), jnp.int32),
            jax.ShapeDtypeStruct((kp, nq_pad), jnp.float32),
        ],
    )(lohi, qs, ks, q2, k2, qb, kb)
    return idx[:k_sel, :nq].T, dist[:k_sel, :nq].T


def _sc_gather(table, idx):
    B = idx.shape[0]
    NW = 32
    bpw = B // NW
    CH = 128
    n_chunks = bpw // CH
    mesh = plsc.VectorSubcoreMesh(core_axis_name="c", subcore_axis_name="s")

    @functools.partial(
        pl.kernel, mesh=mesh,
        compiler_params=pltpu.CompilerParams(use_tc_tiling_on_sc=False),
        out_type=jax.ShapeDtypeStruct((B, D), jnp.float32),
        scratch_types=[
            pltpu.VMEM((bpw,), jnp.int32),
            pltpu.VMEM((bpw, D), jnp.float32),
            pltpu.SemaphoreType.DMA,
        ],
    )
    def gk(table_hbm, idx_hbm, out_hbm, idx_v, rows_v, sem):
        wid = lax.axis_index("s") * 2 + lax.axis_index("c")
        base = wid * bpw
        pltpu.sync_copy(idx_hbm.at[pl.ds(base, bpw)], idx_v)

        def fire(c, carry):
            off = c * CH
            pltpu.async_copy(table_hbm.at[idx_v.at[pl.ds(off, CH)]],
                             rows_v.at[pl.ds(off, CH)], sem)
            return carry

        lax.fori_loop(0, n_chunks, fire, 0)
        pltpu.make_async_copy(table_hbm.at[idx_v], rows_v, sem).wait()
        pltpu.sync_copy(rows_v, out_hbm.at[pl.ds(base, bpw)])

    return gk(table, idx)


def _layer_body(x_ref, g_ref, dd_ref, a1_ref, bm_ref, c1_ref, b1_ref,
                w2t_ref, b2_ref, gnw_ref, gnb_ref, m8_ref, out_ref):
    x = x_ref[...]
    xi = jnp.dot(x, a1_ref[...],
                 preferred_element_type=jnp.float32)
    c1 = c1_ref[...]
    b1 = b1_ref[...]
    w2t = w2t_ref[...]
    acc = jnp.zeros_like(x)
    for k in range(K):
        gk = g_ref[k]
        h = xi + jnp.dot(gk, bm_ref[...],
                         preferred_element_type=jnp.float32)
        h = h + dd_ref[:, k:k + 1] * c1 + b1
        h = jnp.maximum(h, 0.2 * h)
        acc = acc + jnp.dot(h, w2t, preferred_element_type=jnp.float32)
    msum = acc + jnp.float32(K) * b2_ref[...]
    mu = jnp.dot(msum, m8_ref[...], preferred_element_type=jnp.float32)
    cent = msum - mu
    var = jnp.dot(cent * cent, m8_ref[...],
                  preferred_element_type=jnp.float32)
    y = (cent / jnp.sqrt(var + 1e-5)) * gnw_ref[...] + gnb_ref[...]
    out_ref[...] = x + jnp.maximum(y, 0.2 * y)


def _mp_layer(x, g3, dd, w1, b1, w2, b2, gnw, gnb, nb_rows=512):
    n = x.shape[0]
    a1 = w1[:, :D].T
    bm = w1[:, D:2 * D].T
    c1 = w1[:, 2 * D][None, :]
    m8 = jnp.zeros((D, D), jnp.float32)
    m8 = m8.at[:8, :8].set(1.0 / 8).at[8:, 8:].set(1.0 / 8)

    grid = (n // nb_rows,)
    return pl.pallas_call(
        _layer_body,
        grid=grid,
        in_specs=[
            pl.BlockSpec((nb_rows, D), lambda i: (i, 0)),
            pl.BlockSpec((K, nb_rows, D), lambda i: (0, i, 0)),
            pl.BlockSpec((nb_rows, K), lambda i: (i, 0)),
            pl.BlockSpec((D, H), lambda i: (0, 0)),
            pl.BlockSpec((D, H), lambda i: (0, 0)),
            pl.BlockSpec((1, H), lambda i: (0, 0)),
            pl.BlockSpec((1, H), lambda i: (0, 0)),
            pl.BlockSpec((H, D), lambda i: (0, 0)),
            pl.BlockSpec((1, D), lambda i: (0, 0)),
            pl.BlockSpec((1, D), lambda i: (0, 0)),
            pl.BlockSpec((1, D), lambda i: (0, 0)),
            pl.BlockSpec((D, D), lambda i: (0, 0)),
        ],
        out_specs=pl.BlockSpec((nb_rows, D), lambda i: (i, 0)),
        out_shape=jax.ShapeDtypeStruct((n, D), jnp.float32),
    )(x, g3, dd, a1, bm, c1, b1[None, :], w2.T, b2[None, :],
      gnw[None, :], gnb[None, :], m8)


def _ae3_body(g_ref, dd_ref, a1_ref, bm_ref, c1_ref, b1_ref,
              w2t_ref, b2_ref, gnw_ref, gnb_ref, m8_ref, out_ref):
    x = jnp.ones_like(out_ref)
    m8 = m8_ref[...]
    for i in range(N_LAYERS):
        xi = jnp.dot(x, a1_ref[i], preferred_element_type=jnp.float32)
        c1 = c1_ref[i]
        b1 = b1_ref[i]
        w2t = w2t_ref[i]
        acc = jnp.zeros_like(x)
        for k in range(K):
            h = xi + jnp.dot(g_ref[k], bm_ref[i],
                             preferred_element_type=jnp.float32)
            h = h + dd_ref[:, k:k + 1] * c1 + b1
            h = jnp.maximum(h, 0.2 * h)
            acc = acc + jnp.dot(h, w2t, preferred_element_type=jnp.float32)
        msum = acc + jnp.float32(K) * b2_ref[i]
        mu = jnp.dot(msum, m8, preferred_element_type=jnp.float32)
        cent = msum - mu
        var = jnp.dot(cent * cent, m8, preferred_element_type=jnp.float32)
        y = (cent / jnp.sqrt(var + 1e-5)) * gnw_ref[i] + gnb_ref[i]
        x = x + jnp.maximum(y, 0.2 * y)
    out_ref[...] = x


def _ae3(g3, dd, p, nb_rows=512):
    n = dd.shape[0]
    a1 = jnp.stack([p['ae_w1_' + str(i)][:, :D].T for i in range(N_LAYERS)])
    bm = jnp.stack([p['ae_w1_' + str(i)][:, D:2 * D].T
                    for i in range(N_LAYERS)])
    c1 = jnp.stack([p['ae_w1_' + str(i)][:, 2 * D][None, :]
                    for i in range(N_LAYERS)])
    b1 = jnp.stack([p['ae_b1_' + str(i)][None, :] for i in range(N_LAYERS)])
    w2t = jnp.stack([p['ae_w2_' + str(i)].T for i in range(N_LAYERS)])
    b2 = jnp.stack([p['ae_b2_' + str(i)][None, :] for i in range(N_LAYERS)])
    gnw = jnp.stack([p['ae_gn_w_' + str(i)][None, :]
                     for i in range(N_LAYERS)])
    gnb = jnp.stack([p['ae_gn_b_' + str(i)][None, :]
                     for i in range(N_LAYERS)])
    m8 = jnp.zeros((D, D), jnp.float32)
    m8 = m8.at[:8, :8].set(1.0 / 8).at[8:, 8:].set(1.0 / 8)

    full = lambda s: pl.BlockSpec(s, lambda i: (0,) * len(s))
    return pl.pallas_call(
        _ae3_body,
        grid=(n // nb_rows,),
        in_specs=[
            pl.BlockSpec((K, nb_rows, D), lambda i: (0, i, 0)),
            pl.BlockSpec((nb_rows, K), lambda i: (i, 0)),
            full((N_LAYERS, D, H)), full((N_LAYERS, D, H)),
            full((N_LAYERS, 1, H)), full((N_LAYERS, 1, H)),
            full((N_LAYERS, H, D)), full((N_LAYERS, 1, D)),
            full((N_LAYERS, 1, D)), full((N_LAYERS, 1, D)),
            full((D, D)),
        ],
        out_specs=pl.BlockSpec((nb_rows, D), lambda i: (i, 0)),
        out_shape=jax.ShapeDtypeStruct((n, D), jnp.float32),
    )(g3, dd, a1, bm, c1, b1, w2t, b2, gnw, gnb, m8)


def _tf_body(x_ref, w1t_ref, b1_ref, w2t_ref, b2_ref, out_ref):
    h = jnp.dot(x_ref[...], w1t_ref[...],
                preferred_element_type=jnp.float32) + b1_ref[...]
    h = jnp.maximum(h, 0.2 * h)
    out_ref[...] = jnp.dot(h, w2t_ref[...],
                           preferred_element_type=jnp.float32) + b2_ref[...]


def _tf(x, w1, b1, w2, b2):
    n = x.shape[0]
    return pl.pallas_call(
        _tf_body,
        grid=(1,),
        in_specs=[
            pl.BlockSpec((n, D), lambda i: (0, 0)),
            pl.BlockSpec((D, D), lambda i: (0, 0)),
            pl.BlockSpec((1, D), lambda i: (0, 0)),
            pl.BlockSpec((D, D), lambda i: (0, 0)),
            pl.BlockSpec((1, D), lambda i: (0, 0)),
        ],
        out_specs=pl.BlockSpec((n, D), lambda i: (0, 0)),
        out_shape=jax.ShapeDtypeStruct((n, D), jnp.float32),
    )(x, w1.T, b1[None, :], w2.T, b2[None, :])


def kernel(xyz, atom_xyz, atom_features, batch, atom_batch, params):
    p = params
    n_surf, n_atom = xyz.shape[0], atom_xyz.shape[0]
    na_pad = ((n_atom + 511) // 512) * 512
    ns_pad = ((n_surf + 511) // 512) * 512

    af_pad = jnp.pad(atom_features, ((0, na_pad - n_atom), (0, 0)))
    out = _tf(af_pad, p['tf_w1'], p['tf_b1'], p['tf_w2'], p['tf_b2'])

    idx, dist = _knn(atom_xyz, atom_xyz, atom_batch, atom_batch, K + 1)
    idx = idx[:, 1:]
    dd = jnp.pad(dist[:, 1:], ((0, na_pad - n_atom), (0, 0)))
    idx_flat = jnp.pad(idx.T.astype(jnp.int32),
                       ((0, 0), (0, na_pad - n_atom))).reshape(-1)
    for i in range(N_LAYERS):
        g = _sc_gather(out, idx_flat).reshape(K, na_pad, D)
        out = _mp_layer(out, g, dd,
                        p['aa_w1_' + str(i)], p['aa_b1_' + str(i)],
                        p['aa_w2_' + str(i)], p['aa_b2_' + str(i)],
                        p['aa_gn_w_' + str(i)], p['aa_gn_b_' + str(i)])

    idx2, dist2 = _knn(xyz, atom_xyz, batch, atom_batch, K)
    dd2 = jnp.pad(dist2, ((0, ns_pad - n_surf), (0, 0)))
    idx2_flat = jnp.pad(idx2.T.astype(jnp.int32),
                        ((0, 0), (0, ns_pad - n_surf))).reshape(-1)
    g2 = _sc_gather(out, idx2_flat).reshape(K, ns_pad, D)
    pe = _ae3(g2, dd2, p)
    return pe[:n_surf]


# speedup vs baseline: 20.1264x; 1.0759x over previous
import functools

import jax
import jax.numpy as jnp
from jax import lax
from jax.experimental import pallas as pl
from jax.experimental.pallas import tpu as pltpu
from jax.experimental.pallas import tpu_sc as plsc

D = 16
K = 16
N_LAYERS = 3
H = 2 * D + 1
MASKVAL = 1e10
DONEVAL = 3e10


_TW = 512
_IBIG = 1 << 30


def _knn_body(lohi_ref, qs_ref, ks_ref, q2_ref, k2_ref, qb_ref, kb_ref,
              idx_ref, dist_ref, *, k_sel):
    i = pl.program_id(0)
    lo = lohi_ref[0, i]
    hi = lohi_ref[1, i]
    qs = qs_ref[...]
    q2 = q2_ref[...]
    qb = qb_ref[...]

    cds = tuple(jnp.full(q2.shape, MASKVAL, jnp.float32)
                for _ in range(k_sel))
    cis = tuple(jnp.full(q2.shape, p, jnp.int32) for p in range(k_sel))

    def tile_step(t, carry):
        cds, cis = carry
        off = pl.multiple_of(t * _TW, _TW)
        kst = ks_ref[pl.ds(off, _TW), :]
        k2t = k2_ref[pl.ds(off, _TW), :]
        kbt = kb_ref[pl.ds(off, _TW), :]
        cross = lax.dot_general(kst, qs, (((1,), (0,)), ((), ())),
                                preferred_element_type=jnp.float32)
        dt = (q2 + k2t) - 2.0 * cross
        dt = jnp.where(kbt != qb, MASKVAL, dt)
        it = lax.broadcasted_iota(jnp.int32, dt.shape, 0) + off
        ad = jnp.concatenate(list(cds) + [dt], axis=0)
        ai = jnp.concatenate(list(cis) + [it], axis=0)
        ncd, nci = [], []
        for _ in range(k_sel):
            m = jnp.min(ad, axis=0, keepdims=True)
            eq = ad == m
            ii = jnp.min(jnp.where(eq, ai, _IBIG), axis=0, keepdims=True)
            ncd.append(m)
            nci.append(ii)
            ad = jnp.where(eq, DONEVAL, ad)
        return tuple(ncd), tuple(nci)

    cds, cis = lax.fori_loop(lo, hi, tile_step, (cds, cis))
    for p in range(k_sel):
        dist_ref[p:p + 1, :] = cds[p]
        idx_ref[p:p + 1, :] = cis[p]


def _knn(q_xyz, k_xyz, q_batch, k_batch, k_sel, qb_rows=128):
    nq, nk = q_xyz.shape[0], k_xyz.shape[0]
    nq_pad = ((nq + qb_rows - 1) // qb_rows) * qb_rows
    nk_pad = ((nk + _TW - 1) // _TW) * _TW
    nblk = nq_pad // qb_rows
    kp = ((k_sel + 7) // 8) * 8

    qs = jnp.pad(q_xyz.T, ((0, 5), (0, nq_pad - nq)))
    ks = jnp.pad(k_xyz, ((0, nk_pad - nk), (0, 5)))
    q2 = jnp.pad(jnp.sum(q_xyz * q_xyz, axis=1), (0, nq_pad - nq))[None, :]
    k2 = jnp.pad(jnp.sum(k_xyz * k_xyz, axis=1), (0, nk_pad - nk))[:, None]
    qb = jnp.pad(q_batch.astype(jnp.int32), (0, nq_pad - nq),
                 constant_values=1 << 20)[None, :]
    kb = jnp.pad(k_batch.astype(jnp.int32), (0, nk_pad - nk),
                 constant_values=-2)[:, None]

    blo = qb[0, ::qb_rows]
    bhi = qb[0, qb_rows - 1::qb_rows]
    kstart = jnp.searchsorted(k_batch, blo).astype(jnp.int32)
    kend = jnp.searchsorted(k_batch, bhi, side='right').astype(jnp.int32)
    lohi = jnp.stack([kstart // _TW, (kend + _TW - 1) // _TW], axis=0)

    idx, dist = pl.pallas_call(
        functools.partial(_knn_body, k_sel=k_sel),
        grid=(nblk,),
        in_specs=[
            pl.BlockSpec(memory_space=pltpu.SMEM),
            pl.BlockSpec((8, qb_rows), lambda i: (0, i)),
            pl.BlockSpec((nk_pad, 8), lambda i: (0, 0)),
            pl.BlockSpec((1, qb_rows), lambda i: (0, i)),
            pl.BlockSpec((nk_pad, 1), lambda i: (0, 0)),
            pl.BlockSpec((1, qb_rows), lambda i: (0, i)),
            pl.BlockSpec((nk_pad, 1), lambda i: (0, 0)),
        ],
        out_specs=[
            pl.BlockSpec((kp, qb_rows), lambda i: (0, i)),
            pl.BlockSpec((kp, qb_rows), lambda i: (0, i)),
        ],
        out_shape=[
            jax.ShapeDtypeStruct((kp, nq_pad), jnp.int32),
            jax.ShapeDtypeStruct((kp, nq_pad), jnp.float32),
        ],
    )(lohi, qs, ks, q2, k2, qb, kb)
    return idx[:k_sel, :nq].T, dist[:k_sel, :nq].T


def _sc_gather(table, idx):
    B = idx.shape[0]
    NW = 32
    bpw = B // NW
    CH = 128
    n_chunks = bpw // CH
    mesh = plsc.VectorSubcoreMesh(core_axis_name="c", subcore_axis_name="s")

    @functools.partial(
        pl.kernel, mesh=mesh,
        compiler_params=pltpu.CompilerParams(use_tc_tiling_on_sc=False),
        out_type=jax.ShapeDtypeStruct((B, D), jnp.float32),
        scratch_types=[
            pltpu.VMEM((bpw,), jnp.int32),
            pltpu.VMEM((bpw, D), jnp.float32),
            pltpu.SemaphoreType.DMA,
        ],
    )
    def gk(table_hbm, idx_hbm, out_hbm, idx_v, rows_v, sem):
        wid = lax.axis_index("s") * 2 + lax.axis_index("c")
        base = wid * bpw
        pltpu.sync_copy(idx_hbm.at[pl.ds(base, bpw)], idx_v)

        def fire(c, carry):
            off = c * CH
            pltpu.async_copy(table_hbm.at[idx_v.at[pl.ds(off, CH)]],
                             rows_v.at[pl.ds(off, CH)], sem)
            return carry

        lax.fori_loop(0, n_chunks, fire, 0)
        pltpu.make_async_copy(table_hbm.at[idx_v], rows_v, sem).wait()
        pltpu.sync_copy(rows_v, out_hbm.at[pl.ds(base, bpw)])

    return gk(table, idx)


def _layer_body(x_ref, g_ref, dd_ref, a1_ref, bm_ref, c1_ref, b1_ref,
                w2t_ref, b2_ref, gnw_ref, gnb_ref, m8_ref, out_ref):
    x = x_ref[...]
    xi = jnp.dot(x, a1_ref[...],
                 preferred_element_type=jnp.float32)
    c1 = c1_ref[...]
    b1 = b1_ref[...]
    w2t = w2t_ref[...]
    acc = jnp.zeros_like(x)
    for k in range(K):
        gk = g_ref[k]
        h = xi + jnp.dot(gk, bm_ref[...],
                         preferred_element_type=jnp.float32)
        h = h + dd_ref[:, k:k + 1] * c1 + b1
        h = jnp.maximum(h, 0.2 * h)
        acc = acc + jnp.dot(h, w2t, preferred_element_type=jnp.float32)
    msum = acc + jnp.float32(K) * b2_ref[...]
    mu = jnp.dot(msum, m8_ref[...], preferred_element_type=jnp.float32)
    cent = msum - mu
    var = jnp.dot(cent * cent, m8_ref[...],
                  preferred_element_type=jnp.float32)
    y = (cent / jnp.sqrt(var + 1e-5)) * gnw_ref[...] + gnb_ref[...]
    out_ref[...] = x + jnp.maximum(y, 0.2 * y)


def _mp_layer(x, g3, dd, w1, b1, w2, b2, gnw, gnb, nb_rows=512):
    n = x.shape[0]
    a1 = w1[:, :D].T
    bm = w1[:, D:2 * D].T
    c1 = w1[:, 2 * D][None, :]
    m8 = jnp.zeros((D, D), jnp.float32)
    m8 = m8.at[:8, :8].set(1.0 / 8).at[8:, 8:].set(1.0 / 8)

    grid = (n // nb_rows,)
    return pl.pallas_call(
        _layer_body,
        grid=grid,
        in_specs=[
            pl.BlockSpec((nb_rows, D), lambda i: (i, 0)),
            pl.BlockSpec((K, nb_rows, D), lambda i: (0, i, 0)),
            pl.BlockSpec((nb_rows, K), lambda i: (i, 0)),
            pl.BlockSpec((D, H), lambda i: (0, 0)),
            pl.BlockSpec((D, H), lambda i: (0, 0)),
            pl.BlockSpec((1, H), lambda i: (0, 0)),
            pl.BlockSpec((1, H), lambda i: (0, 0)),
            pl.BlockSpec((H, D), lambda i: (0, 0)),
            pl.BlockSpec((1, D), lambda i: (0, 0)),
            pl.BlockSpec((1, D), lambda i: (0, 0)),
            pl.BlockSpec((1, D), lambda i: (0, 0)),
            pl.BlockSpec((D, D), lambda i: (0, 0)),
        ],
        out_specs=pl.BlockSpec((nb_rows, D), lambda i: (i, 0)),
        out_shape=jax.ShapeDtypeStruct((n, D), jnp.float32),
    )(x, g3, dd, a1, bm, c1, b1[None, :], w2.T, b2[None, :],
      gnw[None, :], gnb[None, :], m8)


def _ae3_body(g_ref, dd_ref, a1_ref, bm_ref, c1_ref, b1_ref,
              w2t_ref, b2_ref, gnw_ref, gnb_ref, m8_ref, out_ref):
    x = jnp.ones_like(out_ref)
    m8 = m8_ref[...]
    for i in range(N_LAYERS):
        xi = jnp.dot(x, a1_ref[i], preferred_element_type=jnp.float32)
        c1 = c1_ref[i]
        b1 = b1_ref[i]
        w2t = w2t_ref[i]
        acc = jnp.zeros_like(x)
        for k in range(K):
            h = xi + jnp.dot(g_ref[k], bm_ref[i],
                             preferred_element_type=jnp.float32)
            h = h + dd_ref[:, k:k + 1] * c1 + b1
            h = jnp.maximum(h, 0.2 * h)
            acc = acc + jnp.dot(h, w2t, preferred_element_type=jnp.float32)
        msum = acc + jnp.float32(K) * b2_ref[i]
        mu = jnp.dot(msum, m8, preferred_element_type=jnp.float32)
        cent = msum - mu
        var = jnp.dot(cent * cent, m8, preferred_element_type=jnp.float32)
        y = (cent / jnp.sqrt(var + 1e-5)) * gnw_ref[i] + gnb_ref[i]
        x = x + jnp.maximum(y, 0.2 * y)
    out_ref[...] = x


def _ae3(g3, dd, p, nb_rows=512):
    n = dd.shape[0]
    a1 = jnp.stack([p['ae_w1_' + str(i)][:, :D].T for i in range(N_LAYERS)])
    bm = jnp.stack([p['ae_w1_' + str(i)][:, D:2 * D].T
                    for i in range(N_LAYERS)])
    c1 = jnp.stack([p['ae_w1_' + str(i)][:, 2 * D][None, :]
                    for i in range(N_LAYERS)])
    b1 = jnp.stack([p['ae_b1_' + str(i)][None, :] for i in range(N_LAYERS)])
    w2t = jnp.stack([p['ae_w2_' + str(i)].T for i in range(N_LAYERS)])
    b2 = jnp.stack([p['ae_b2_' + str(i)][None, :] for i in range(N_LAYERS)])
    gnw = jnp.stack([p['ae_gn_w_' + str(i)][None, :]
                     for i in range(N_LAYERS)])
    gnb = jnp.stack([p['ae_gn_b_' + str(i)][None, :]
                     for i in range(N_LAYERS)])
    m8 = jnp.zeros((D, D), jnp.float32)
    m8 = m8.at[:8, :8].set(1.0 / 8).at[8:, 8:].set(1.0 / 8)

    full = lambda s: pl.BlockSpec(s, lambda i: (0,) * len(s))
    return pl.pallas_call(
        _ae3_body,
        grid=(n // nb_rows,),
        in_specs=[
            pl.BlockSpec((K, nb_rows, D), lambda i: (0, i, 0)),
            pl.BlockSpec((nb_rows, K), lambda i: (i, 0)),
            full((N_LAYERS, D, H)), full((N_LAYERS, D, H)),
            full((N_LAYERS, 1, H)), full((N_LAYERS, 1, H)),
            full((N_LAYERS, H, D)), full((N_LAYERS, 1, D)),
            full((N_LAYERS, 1, D)), full((N_LAYERS, 1, D)),
            full((D, D)),
        ],
        out_specs=pl.BlockSpec((nb_rows, D), lambda i: (i, 0)),
        out_shape=jax.ShapeDtypeStruct((n, D), jnp.float32),
    )(g3, dd, a1, bm, c1, b1, w2t, b2, gnw, gnb, m8)


def _tf_body(x_ref, w1t_ref, b1_ref, w2t_ref, b2_ref, out_ref):
    h = jnp.dot(x_ref[...], w1t_ref[...],
                preferred_element_type=jnp.float32) + b1_ref[...]
    h = jnp.maximum(h, 0.2 * h)
    out_ref[...] = jnp.dot(h, w2t_ref[...],
                           preferred_element_type=jnp.float32) + b2_ref[...]


def _tf(x, w1, b1, w2, b2):
    n = x.shape[0]
    return pl.pallas_call(
        _tf_body,
        grid=(1,),
        in_specs=[
            pl.BlockSpec((n, D), lambda i: (0, 0)),
            pl.BlockSpec((D, D), lambda i: (0, 0)),
            pl.BlockSpec((1, D), lambda i: (0, 0)),
            pl.BlockSpec((D, D), lambda i: (0, 0)),
            pl.BlockSpec((1, D), lambda i: (0, 0)),
        ],
        out_specs=pl.BlockSpec((n, D), lambda i: (0, 0)),
        out_shape=jax.ShapeDtypeStruct((n, D), jnp.float32),
    )(x, w1.T, b1[None, :], w2.T, b2[None, :])


def kernel(xyz, atom_xyz, atom_features, batch, atom_batch, params):
    p = params
    n_surf, n_atom = xyz.shape[0], atom_xyz.shape[0]
    na_pad = ((n_atom + 511) // 512) * 512
    ns_pad = ((n_surf + 511) // 512) * 512

    af_pad = jnp.pad(atom_features, ((0, na_pad - n_atom), (0, 0)))
    out = _tf(af_pad, p['tf_w1'], p['tf_b1'], p['tf_w2'], p['tf_b2'])

    idx, dist = _knn(atom_xyz, atom_xyz, atom_batch, atom_batch, K + 1)
    idx = idx[:, 1:]
    dd = jnp.pad(dist[:, 1:], ((0, na_pad - n_atom), (0, 0)))
    idx_flat = jnp.pad(idx.T.astype(jnp.int32),
                       ((0, 0), (0, na_pad - n_atom))).reshape(-1)
    for i in range(N_LAYERS):
        g = _sc_gather(out, idx_flat).reshape(K, na_pad, D)
        out = _mp_layer(out, g, dd,
                        p['aa_w1_' + str(i)], p['aa_b1_' + str(i)],
                        p['aa_w2_' + str(i)], p['aa_b2_' + str(i)],
                        p['aa_gn_w_' + str(i)], p['aa_gn_b_' + str(i)])

    idx2, dist2 = _knn(xyz, atom_xyz, batch, atom_batch, K)
    dd2 = jnp.pad(dist2, ((0, ns_pad - n_surf), (0, 0)))
    idx2_flat = jnp.pad(idx2.T.astype(jnp.int32),
                        ((0, 0), (0, ns_pad - n_surf))).reshape(-1)
    g2 = _sc_gather(out, idx2_flat).reshape(K, ns_pad, D)
    pe = _ae3(g2, dd2, p)
    return pe[:n_surf]


# speedup vs baseline: 21.8957x; 1.0879x over previous
import functools

import jax
import jax.numpy as jnp
from jax import lax
from jax.experimental import pallas as pl
from jax.experimental.pallas import tpu as pltpu
from jax.experimental.pallas import tpu_sc as plsc

D = 16
K = 16
N_LAYERS = 3
H = 2 * D + 1
MASKVAL = 1e10
DONEVAL = 3e10


_TW = 256
_IBIG = 1 << 30


def _knn_body(lohi_ref, qs_ref, ks_ref, q2_ref, k2_ref, qb_ref, kb_ref,
              idx_ref, dist_ref, *, k_sel):
    i = pl.program_id(0)
    lo = lohi_ref[0, i]
    n_tiles = lohi_ref[1, i]
    qs = qs_ref[...]
    q2 = q2_ref[...]
    qb = qb_ref[...]

    cds = tuple(jnp.full(q2.shape, MASKVAL, jnp.float32)
                for _ in range(k_sel))
    cis = tuple(jnp.full(q2.shape, p, jnp.int32) for p in range(k_sel))

    def tile_step(t, carry):
        cds, cis = carry
        off = pl.multiple_of(lo + t * _TW, 8)
        kst = ks_ref[pl.ds(off, _TW), :]
        k2t = k2_ref[pl.ds(off, _TW), :]
        kbt = kb_ref[pl.ds(off, _TW), :]
        cross = lax.dot_general(kst, qs, (((1,), (0,)), ((), ())),
                                preferred_element_type=jnp.float32)
        dt = (q2 + k2t) - 2.0 * cross
        dt = jnp.where(kbt != qb, MASKVAL, dt)
        it = lax.broadcasted_iota(jnp.int32, dt.shape, 0) + off
        ad = jnp.concatenate(list(cds) + [dt], axis=0)
        ai = jnp.concatenate(list(cis) + [it], axis=0)
        ncd, nci = [], []
        for _ in range(k_sel):
            m = jnp.min(ad, axis=0, keepdims=True)
            eq = ad == m
            ii = jnp.min(jnp.where(eq, ai, _IBIG), axis=0, keepdims=True)
            ncd.append(m)
            nci.append(ii)
            ad = jnp.where(eq, DONEVAL, ad)
        return tuple(ncd), tuple(nci)

    cds, cis = lax.fori_loop(0, n_tiles, tile_step, (cds, cis))
    for p in range(k_sel):
        dist_ref[p:p + 1, :] = cds[p]
        idx_ref[p:p + 1, :] = cis[p]


def _knn(q_xyz, k_xyz, q_batch, k_batch, k_sel, qb_rows=128):
    nq, nk = q_xyz.shape[0], k_xyz.shape[0]
    nq_pad = ((nq + qb_rows - 1) // qb_rows) * qb_rows
    nk_pad = ((nk + _TW - 1) // _TW) * _TW + _TW
    nblk = nq_pad // qb_rows
    kp = ((k_sel + 7) // 8) * 8

    qs = jnp.pad(q_xyz.T, ((0, 5), (0, nq_pad - nq)))
    ks = jnp.pad(k_xyz, ((0, nk_pad - nk), (0, 5)))
    q2 = jnp.pad(jnp.sum(q_xyz * q_xyz, axis=1), (0, nq_pad - nq))[None, :]
    k2 = jnp.pad(jnp.sum(k_xyz * k_xyz, axis=1), (0, nk_pad - nk))[:, None]
    qb = jnp.pad(q_batch.astype(jnp.int32), (0, nq_pad - nq),
                 constant_values=1 << 20)[None, :]
    kb = jnp.pad(k_batch.astype(jnp.int32), (0, nk_pad - nk),
                 constant_values=-2)[:, None]

    blo = qb[0, ::qb_rows]
    bhi = qb[0, qb_rows - 1::qb_rows]
    kstart = jnp.searchsorted(k_batch, blo).astype(jnp.int32)
    kend = jnp.searchsorted(k_batch, bhi, side='right').astype(jnp.int32)
    lo = (kstart // 8) * 8
    ntiles = (kend - lo + _TW - 1) // _TW
    lohi = jnp.stack([lo, ntiles], axis=0)

    idx, dist = pl.pallas_call(
        functools.partial(_knn_body, k_sel=k_sel),
        grid=(nblk,),
        in_specs=[
            pl.BlockSpec(memory_space=pltpu.SMEM),
            pl.BlockSpec((8, qb_rows), lambda i: (0, i)),
            pl.BlockSpec((nk_pad, 8), lambda i: (0, 0)),
            pl.BlockSpec((1, qb_rows), lambda i: (0, i)),
            pl.BlockSpec((nk_pad, 1), lambda i: (0, 0)),
            pl.BlockSpec((1, qb_rows), lambda i: (0, i)),
            pl.BlockSpec((nk_pad, 1), lambda i: (0, 0)),
        ],
        out_specs=[
            pl.BlockSpec((kp, qb_rows), lambda i: (0, i)),
            pl.BlockSpec((kp, qb_rows), lambda i: (0, i)),
        ],
        out_shape=[
            jax.ShapeDtypeStruct((kp, nq_pad), jnp.int32),
            jax.ShapeDtypeStruct((kp, nq_pad), jnp.float32),
        ],
    )(lohi, qs, ks, q2, k2, qb, kb)
    return idx[:k_sel, :nq].T, dist[:k_sel, :nq].T


def _sc_gather(table, idx):
    B = idx.shape[0]
    NW = 32
    bpw = B // NW
    CH = 128
    n_chunks = bpw // CH
    mesh = plsc.VectorSubcoreMesh(core_axis_name="c", subcore_axis_name="s")

    @functools.partial(
        pl.kernel, mesh=mesh,
        compiler_params=pltpu.CompilerParams(use_tc_tiling_on_sc=False),
        out_type=jax.ShapeDtypeStruct((B, D), jnp.float32),
        scratch_types=[
            pltpu.VMEM((bpw,), jnp.int32),
            pltpu.VMEM((bpw, D), jnp.float32),
            pltpu.SemaphoreType.DMA,
        ],
    )
    def gk(table_hbm, idx_hbm, out_hbm, idx_v, rows_v, sem):
        wid = lax.axis_index("s") * 2 + lax.axis_index("c")
        base = wid * bpw
        pltpu.sync_copy(idx_hbm.at[pl.ds(base, bpw)], idx_v)

        def fire(c, carry):
            off = c * CH
            pltpu.async_copy(table_hbm.at[idx_v.at[pl.ds(off, CH)]],
                             rows_v.at[pl.ds(off, CH)], sem)
            return carry

        lax.fori_loop(0, n_chunks, fire, 0)
        pltpu.make_async_copy(table_hbm.at[idx_v], rows_v, sem).wait()
        pltpu.sync_copy(rows_v, out_hbm.at[pl.ds(base, bpw)])

    return gk(table, idx)


def _layer_body(x_ref, g_ref, dd_ref, a1_ref, bm_ref, c1_ref, b1_ref,
                w2t_ref, b2_ref, gnw_ref, gnb_ref, m8_ref, out_ref):
    x = x_ref[...]
    xi = jnp.dot(x, a1_ref[...],
                 preferred_element_type=jnp.float32)
    c1 = c1_ref[...]
    b1 = b1_ref[...]
    w2t = w2t_ref[...]
    acc = jnp.zeros_like(x)
    for k in range(K):
        gk = g_ref[k]
        h = xi + jnp.dot(gk, bm_ref[...],
                         preferred_element_type=jnp.float32)
        h = h + dd_ref[:, k:k + 1] * c1 + b1
        h = jnp.maximum(h, 0.2 * h)
        acc = acc + jnp.dot(h, w2t, preferred_element_type=jnp.float32)
    msum = acc + jnp.float32(K) * b2_ref[...]
    mu = jnp.dot(msum, m8_ref[...], preferred_element_type=jnp.float32)
    cent = msum - mu
    var = jnp.dot(cent * cent, m8_ref[...],
                  preferred_element_type=jnp.float32)
    y = (cent / jnp.sqrt(var + 1e-5)) * gnw_ref[...] + gnb_ref[...]
    out_ref[...] = x + jnp.maximum(y, 0.2 * y)


def _mp_layer(x, g3, dd, w1, b1, w2, b2, gnw, gnb, nb_rows=512):
    n = x.shape[0]
    a1 = w1[:, :D].T
    bm = w1[:, D:2 * D].T
    c1 = w1[:, 2 * D][None, :]
    m8 = jnp.zeros((D, D), jnp.float32)
    m8 = m8.at[:8, :8].set(1.0 / 8).at[8:, 8:].set(1.0 / 8)

    grid = (n // nb_rows,)
    return pl.pallas_call(
        _layer_body,
        grid=grid,
        in_specs=[
            pl.BlockSpec((nb_rows, D), lambda i: (i, 0)),
            pl.BlockSpec((K, nb_rows, D), lambda i: (0, i, 0)),
            pl.BlockSpec((nb_rows, K), lambda i: (i, 0)),
            pl.BlockSpec((D, H), lambda i: (0, 0)),
            pl.BlockSpec((D, H), lambda i: (0, 0)),
            pl.BlockSpec((1, H), lambda i: (0, 0)),
            pl.BlockSpec((1, H), lambda i: (0, 0)),
            pl.BlockSpec((H, D), lambda i: (0, 0)),
            pl.BlockSpec((1, D), lambda i: (0, 0)),
            pl.BlockSpec((1, D), lambda i: (0, 0)),
            pl.BlockSpec((1, D), lambda i: (0, 0)),
            pl.BlockSpec((D, D), lambda i: (0, 0)),
        ],
        out_specs=pl.BlockSpec((nb_rows, D), lambda i: (i, 0)),
        out_shape=jax.ShapeDtypeStruct((n, D), jnp.float32),
    )(x, g3, dd, a1, bm, c1, b1[None, :], w2.T, b2[None, :],
      gnw[None, :], gnb[None, :], m8)


def _ae3_body(g_ref, dd_ref, a1_ref, bm_ref, c1_ref, b1_ref,
              w2t_ref, b2_ref, gnw_ref, gnb_ref, m8_ref, out_ref):
    x = jnp.ones_like(out_ref)
    m8 = m8_ref[...]
    for i in range(N_LAYERS):
        xi = jnp.dot(x, a1_ref[i], preferred_element_type=jnp.float32)
        c1 = c1_ref[i]
        b1 = b1_ref[i]
        w2t = w2t_ref[i]
        acc = jnp.zeros_like(x)
        for k in range(K):
            h = xi + jnp.dot(g_ref[k], bm_ref[i],
                             preferred_element_type=jnp.float32)
            h = h + dd_ref[:, k:k + 1] * c1 + b1
            h = jnp.maximum(h, 0.2 * h)
            acc = acc + jnp.dot(h, w2t, preferred_element_type=jnp.float32)
        msum = acc + jnp.float32(K) * b2_ref[i]
        mu = jnp.dot(msum, m8, preferred_element_type=jnp.float32)
        cent = msum - mu
        var = jnp.dot(cent * cent, m8, preferred_element_type=jnp.float32)
        y = (cent / jnp.sqrt(var + 1e-5)) * gnw_ref[i] + gnb_ref[i]
        x = x + jnp.maximum(y, 0.2 * y)
    out_ref[...] = x


def _ae3(g3, dd, p, nb_rows=512):
    n = dd.shape[0]
    a1 = jnp.stack([p['ae_w1_' + str(i)][:, :D].T for i in range(N_LAYERS)])
    bm = jnp.stack([p['ae_w1_' + str(i)][:, D:2 * D].T
                    for i in range(N_LAYERS)])
    c1 = jnp.stack([p['ae_w1_' + str(i)][:, 2 * D][None, :]
                    for i in range(N_LAYERS)])
    b1 = jnp.stack([p['ae_b1_' + str(i)][None, :] for i in range(N_LAYERS)])
    w2t = jnp.stack([p['ae_w2_' + str(i)].T for i in range(N_LAYERS)])
    b2 = jnp.stack([p['ae_b2_' + str(i)][None, :] for i in range(N_LAYERS)])
    gnw = jnp.stack([p['ae_gn_w_' + str(i)][None, :]
                     for i in range(N_LAYERS)])
    gnb = jnp.stack([p['ae_gn_b_' + str(i)][None, :]
                     for i in range(N_LAYERS)])
    m8 = jnp.zeros((D, D), jnp.float32)
    m8 = m8.at[:8, :8].set(1.0 / 8).at[8:, 8:].set(1.0 / 8)

    full = lambda s: pl.BlockSpec(s, lambda i: (0,) * len(s))
    return pl.pallas_call(
        _ae3_body,
        grid=(n // nb_rows,),
        in_specs=[
            pl.BlockSpec((K, nb_rows, D), lambda i: (0, i, 0)),
            pl.BlockSpec((nb_rows, K), lambda i: (i, 0)),
            full((N_LAYERS, D, H)), full((N_LAYERS, D, H)),
            full((N_LAYERS, 1, H)), full((N_LAYERS, 1, H)),
            full((N_LAYERS, H, D)), full((N_LAYERS, 1, D)),
            full((N_LAYERS, 1, D)), full((N_LAYERS, 1, D)),
            full((D, D)),
        ],
        out_specs=pl.BlockSpec((nb_rows, D), lambda i: (i, 0)),
        out_shape=jax.ShapeDtypeStruct((n, D), jnp.float32),
    )(g3, dd, a1, bm, c1, b1, w2t, b2, gnw, gnb, m8)


def _tf_body(x_ref, w1t_ref, b1_ref, w2t_ref, b2_ref, out_ref):
    h = jnp.dot(x_ref[...], w1t_ref[...],
                preferred_element_type=jnp.float32) + b1_ref[...]
    h = jnp.maximum(h, 0.2 * h)
    out_ref[...] = jnp.dot(h, w2t_ref[...],
                           preferred_element_type=jnp.float32) + b2_ref[...]


def _tf(x, w1, b1, w2, b2):
    n = x.shape[0]
    return pl.pallas_call(
        _tf_body,
        grid=(1,),
        in_specs=[
            pl.BlockSpec((n, D), lambda i: (0, 0)),
            pl.BlockSpec((D, D), lambda i: (0, 0)),
            pl.BlockSpec((1, D), lambda i: (0, 0)),
            pl.BlockSpec((D, D), lambda i: (0, 0)),
            pl.BlockSpec((1, D), lambda i: (0, 0)),
        ],
        out_specs=pl.BlockSpec((n, D), lambda i: (0, 0)),
        out_shape=jax.ShapeDtypeStruct((n, D), jnp.float32),
    )(x, w1.T, b1[None, :], w2.T, b2[None, :])


def kernel(xyz, atom_xyz, atom_features, batch, atom_batch, params):
    p = params
    n_surf, n_atom = xyz.shape[0], atom_xyz.shape[0]
    na_pad = ((n_atom + 511) // 512) * 512
    ns_pad = ((n_surf + 511) // 512) * 512

    af_pad = jnp.pad(atom_features, ((0, na_pad - n_atom), (0, 0)))
    out = _tf(af_pad, p['tf_w1'], p['tf_b1'], p['tf_w2'], p['tf_b2'])

    idx, dist = _knn(atom_xyz, atom_xyz, atom_batch, atom_batch, K + 1)
    idx = idx[:, 1:]
    dd = jnp.pad(dist[:, 1:], ((0, na_pad - n_atom), (0, 0)))
    idx_flat = jnp.pad(idx.T.astype(jnp.int32),
                       ((0, 0), (0, na_pad - n_atom))).reshape(-1)
    for i in range(N_LAYERS):
        g = _sc_gather(out, idx_flat).reshape(K, na_pad, D)
        out = _mp_layer(out, g, dd,
                        p['aa_w1_' + str(i)], p['aa_b1_' + str(i)],
                        p['aa_w2_' + str(i)], p['aa_b2_' + str(i)],
                        p['aa_gn_w_' + str(i)], p['aa_gn_b_' + str(i)])

    idx2, dist2 = _knn(xyz, atom_xyz, batch, atom_batch, K)
    dd2 = jnp.pad(dist2, ((0, ns_pad - n_surf), (0, 0)))
    idx2_flat = jnp.pad(idx2.T.astype(jnp.int32),
                        ((0, 0), (0, ns_pad - n_surf))).reshape(-1)
    g2 = _sc_gather(out, idx2_flat).reshape(K, ns_pad, D)
    pe = _ae3(g2, dd2, p)
    return pe[:n_surf]


# speedup vs baseline: 22.7154x; 1.0374x over previous
import functools

import jax
import jax.numpy as jnp
from jax import lax
from jax.experimental import pallas as pl
from jax.experimental.pallas import tpu as pltpu
from jax.experimental.pallas import tpu_sc as plsc

D = 16
K = 16
N_LAYERS = 3
H = 2 * D + 1
MASKVAL = 1e10
DONEVAL = 3e10


_TW = 256
_IBIG = 1 << 30


def _knn_body(lohi_ref, qs_ref, ks_ref, q2_ref, k2_ref, qb_ref, kb_ref,
              idx_ref, dist_ref, *, k_sel):
    i = pl.program_id(0)
    lo = lohi_ref[0, i]
    n_tiles = lohi_ref[1, i]
    qs = qs_ref[...]
    q2 = q2_ref[...]
    qb = qb_ref[...]

    cds = tuple(jnp.full(q2.shape, MASKVAL, jnp.float32)
                for _ in range(k_sel))
    cis = tuple(jnp.full(q2.shape, p, jnp.int32) for p in range(k_sel))

    def tile_step(t, carry):
        cds, cis = carry
        off = pl.multiple_of(lo + t * _TW, 8)
        kst = ks_ref[pl.ds(off, _TW), :]
        k2t = k2_ref[pl.ds(off, _TW), :]
        kbt = kb_ref[pl.ds(off, _TW), :]
        cross = lax.dot_general(kst, qs, (((1,), (0,)), ((), ())),
                                preferred_element_type=jnp.float32)
        dt = (q2 + k2t) - 2.0 * cross
        dt = jnp.where(kbt != qb, MASKVAL, dt)
        it = lax.broadcasted_iota(jnp.int32, dt.shape, 0) + off
        ad = jnp.concatenate(list(cds) + [dt], axis=0)
        ai = jnp.concatenate(list(cis) + [it], axis=0)
        ncd, nci = [], []
        for _ in range(k_sel):
            m = jnp.min(ad, axis=0, keepdims=True)
            eq = ad == m
            ii = jnp.min(jnp.where(eq, ai, _IBIG), axis=0, keepdims=True)
            ncd.append(m)
            nci.append(ii)
            ad = jnp.where(eq, DONEVAL, ad)
        return tuple(ncd), tuple(nci)

    cds, cis = lax.fori_loop(0, n_tiles, tile_step, (cds, cis))
    for p in range(k_sel):
        dist_ref[p:p + 1, :] = cds[p]
        idx_ref[p:p + 1, :] = cis[p]


def _knn(q_xyz, k_xyz, q_batch, k_batch, k_sel, qb_rows=256):
    nq, nk = q_xyz.shape[0], k_xyz.shape[0]
    nq_pad = ((nq + qb_rows - 1) // qb_rows) * qb_rows
    nk_pad = ((nk + _TW - 1) // _TW) * _TW + _TW
    nblk = nq_pad // qb_rows
    kp = ((k_sel + 7) // 8) * 8

    qs = jnp.pad(q_xyz.T, ((0, 5), (0, nq_pad - nq)))
    ks = jnp.pad(k_xyz, ((0, nk_pad - nk), (0, 5)))
    q2 = jnp.pad(jnp.sum(q_xyz * q_xyz, axis=1), (0, nq_pad - nq))[None, :]
    k2 = jnp.pad(jnp.sum(k_xyz * k_xyz, axis=1), (0, nk_pad - nk))[:, None]
    qb = jnp.pad(q_batch.astype(jnp.int32), (0, nq_pad - nq),
                 constant_values=1 << 20)[None, :]
    kb = jnp.pad(k_batch.astype(jnp.int32), (0, nk_pad - nk),
                 constant_values=-2)[:, None]

    blo = qb[0, ::qb_rows]
    bhi = qb[0, qb_rows - 1::qb_rows]
    kstart = jnp.searchsorted(k_batch, blo).astype(jnp.int32)
    kend = jnp.searchsorted(k_batch, bhi, side='right').astype(jnp.int32)
    lo = (kstart // 8) * 8
    ntiles = (kend - lo + _TW - 1) // _TW
    lohi = jnp.stack([lo, ntiles], axis=0)

    idx, dist = pl.pallas_call(
        functools.partial(_knn_body, k_sel=k_sel),
        grid=(nblk,),
        in_specs=[
            pl.BlockSpec(memory_space=pltpu.SMEM),
            pl.BlockSpec((8, qb_rows), lambda i: (0, i)),
            pl.BlockSpec((nk_pad, 8), lambda i: (0, 0)),
            pl.BlockSpec((1, qb_rows), lambda i: (0, i)),
            pl.BlockSpec((nk_pad, 1), lambda i: (0, 0)),
            pl.BlockSpec((1, qb_rows), lambda i: (0, i)),
            pl.BlockSpec((nk_pad, 1), lambda i: (0, 0)),
        ],
        out_specs=[
            pl.BlockSpec((kp, qb_rows), lambda i: (0, i)),
            pl.BlockSpec((kp, qb_rows), lambda i: (0, i)),
        ],
        out_shape=[
            jax.ShapeDtypeStruct((kp, nq_pad), jnp.int32),
            jax.ShapeDtypeStruct((kp, nq_pad), jnp.float32),
        ],
    )(lohi, qs, ks, q2, k2, qb, kb)
    return idx[:k_sel, :nq].T, dist[:k_sel, :nq].T


def _sc_gather(table, idx):
    B = idx.shape[0]
    NW = 32
    bpw = B // NW
    CH = 128
    n_chunks = bpw // CH
    mesh = plsc.VectorSubcoreMesh(core_axis_name="c", subcore_axis_name="s")

    @functools.partial(
        pl.kernel, mesh=mesh,
        compiler_params=pltpu.CompilerParams(use_tc_tiling_on_sc=False),
        out_type=jax.ShapeDtypeStruct((B, D), jnp.float32),
        scratch_types=[
            pltpu.VMEM((bpw,), jnp.int32),
            pltpu.VMEM((bpw, D), jnp.float32),
            pltpu.SemaphoreType.DMA,
        ],
    )
    def gk(table_hbm, idx_hbm, out_hbm, idx_v, rows_v, sem):
        wid = lax.axis_index("s") * 2 + lax.axis_index("c")
        base = wid * bpw
        pltpu.sync_copy(idx_hbm.at[pl.ds(base, bpw)], idx_v)

        def fire(c, carry):
            off = c * CH
            pltpu.async_copy(table_hbm.at[idx_v.at[pl.ds(off, CH)]],
                             rows_v.at[pl.ds(off, CH)], sem)
            return carry

        lax.fori_loop(0, n_chunks, fire, 0)
        pltpu.make_async_copy(table_hbm.at[idx_v], rows_v, sem).wait()
        pltpu.sync_copy(rows_v, out_hbm.at[pl.ds(base, bpw)])

    return gk(table, idx)


def _layer_body(x_ref, g_ref, dd_ref, a1_ref, bm_ref, c1_ref, b1_ref,
                w2t_ref, b2_ref, gnw_ref, gnb_ref, m8_ref, out_ref):
    x = x_ref[...]
    xi = jnp.dot(x, a1_ref[...],
                 preferred_element_type=jnp.float32)
    c1 = c1_ref[...]
    b1 = b1_ref[...]
    w2t = w2t_ref[...]
    acc = jnp.zeros_like(x)
    for k in range(K):
        gk = g_ref[k]
        h = xi + jnp.dot(gk, bm_ref[...],
                         preferred_element_type=jnp.float32)
        h = h + dd_ref[:, k:k + 1] * c1 + b1
        h = jnp.maximum(h, 0.2 * h)
        acc = acc + jnp.dot(h, w2t, preferred_element_type=jnp.float32)
    msum = acc + jnp.float32(K) * b2_ref[...]
    mu = jnp.dot(msum, m8_ref[...], preferred_element_type=jnp.float32)
    cent = msum - mu
    var = jnp.dot(cent * cent, m8_ref[...],
                  preferred_element_type=jnp.float32)
    y = (cent / jnp.sqrt(var + 1e-5)) * gnw_ref[...] + gnb_ref[...]
    out_ref[...] = x + jnp.maximum(y, 0.2 * y)


def _mp_layer(x, g3, dd, w1, b1, w2, b2, gnw, gnb, nb_rows=512):
    n = x.shape[0]
    a1 = w1[:, :D].T
    bm = w1[:, D:2 * D].T
    c1 = w1[:, 2 * D][None, :]
    m8 = jnp.zeros((D, D), jnp.float32)
    m8 = m8.at[:8, :8].set(1.0 / 8).at[8:, 8:].set(1.0 / 8)

    grid = (n // nb_rows,)
    return pl.pallas_call(
        _layer_body,
        grid=grid,
        in_specs=[
            pl.BlockSpec((nb_rows, D), lambda i: (i, 0)),
            pl.BlockSpec((K, nb_rows, D), lambda i: (0, i, 0)),
            pl.BlockSpec((nb_rows, K), lambda i: (i, 0)),
            pl.BlockSpec((D, H), lambda i: (0, 0)),
            pl.BlockSpec((D, H), lambda i: (0, 0)),
            pl.BlockSpec((1, H), lambda i: (0, 0)),
            pl.BlockSpec((1, H), lambda i: (0, 0)),
            pl.BlockSpec((H, D), lambda i: (0, 0)),
            pl.BlockSpec((1, D), lambda i: (0, 0)),
            pl.BlockSpec((1, D), lambda i: (0, 0)),
            pl.BlockSpec((1, D), lambda i: (0, 0)),
            pl.BlockSpec((D, D), lambda i: (0, 0)),
        ],
        out_specs=pl.BlockSpec((nb_rows, D), lambda i: (i, 0)),
        out_shape=jax.ShapeDtypeStruct((n, D), jnp.float32),
    )(x, g3, dd, a1, bm, c1, b1[None, :], w2.T, b2[None, :],
      gnw[None, :], gnb[None, :], m8)


def _ae3_body(g_ref, dd_ref, a1_ref, bm_ref, c1_ref, b1_ref,
              w2t_ref, b2_ref, gnw_ref, gnb_ref, m8_ref, out_ref):
    x = jnp.ones_like(out_ref)
    m8 = m8_ref[...]
    for i in range(N_LAYERS):
        xi = jnp.dot(x, a1_ref[i], preferred_element_type=jnp.float32)
        c1 = c1_ref[i]
        b1 = b1_ref[i]
        w2t = w2t_ref[i]
        acc = jnp.zeros_like(x)
        for k in range(K):
            h = xi + jnp.dot(g_ref[k], bm_ref[i],
                             preferred_element_type=jnp.float32)
            h = h + dd_ref[:, k:k + 1] * c1 + b1
            h = jnp.maximum(h, 0.2 * h)
            acc = acc + jnp.dot(h, w2t, preferred_element_type=jnp.float32)
        msum = acc + jnp.float32(K) * b2_ref[i]
        mu = jnp.dot(msum, m8, preferred_element_type=jnp.float32)
        cent = msum - mu
        var = jnp.dot(cent * cent, m8, preferred_element_type=jnp.float32)
        y = (cent / jnp.sqrt(var + 1e-5)) * gnw_ref[i] + gnb_ref[i]
        x = x + jnp.maximum(y, 0.2 * y)
    out_ref[...] = x


def _ae3(g3, dd, p, nb_rows=512):
    n = dd.shape[0]
    a1 = jnp.stack([p['ae_w1_' + str(i)][:, :D].T for i in range(N_LAYERS)])
    bm = jnp.stack([p['ae_w1_' + str(i)][:, D:2 * D].T
                    for i in range(N_LAYERS)])
    c1 = jnp.stack([p['ae_w1_' + str(i)][:, 2 * D][None, :]
                    for i in range(N_LAYERS)])
    b1 = jnp.stack([p['ae_b1_' + str(i)][None, :] for i in range(N_LAYERS)])
    w2t = jnp.stack([p['ae_w2_' + str(i)].T for i in range(N_LAYERS)])
    b2 = jnp.stack([p['ae_b2_' + str(i)][None, :] for i in range(N_LAYERS)])
    gnw = jnp.stack([p['ae_gn_w_' + str(i)][None, :]
                     for i in range(N_LAYERS)])
    gnb = jnp.stack([p['ae_gn_b_' + str(i)][None, :]
                     for i in range(N_LAYERS)])
    m8 = jnp.zeros((D, D), jnp.float32)
    m8 = m8.at[:8, :8].set(1.0 / 8).at[8:, 8:].set(1.0 / 8)

    full = lambda s: pl.BlockSpec(s, lambda i: (0,) * len(s))
    return pl.pallas_call(
        _ae3_body,
        grid=(n // nb_rows,),
        in_specs=[
            pl.BlockSpec((K, nb_rows, D), lambda i: (0, i, 0)),
            pl.BlockSpec((nb_rows, K), lambda i: (i, 0)),
            full((N_LAYERS, D, H)), full((N_LAYERS, D, H)),
            full((N_LAYERS, 1, H)), full((N_LAYERS, 1, H)),
            full((N_LAYERS, H, D)), full((N_LAYERS, 1, D)),
            full((N_LAYERS, 1, D)), full((N_LAYERS, 1, D)),
            full((D, D)),
        ],
        out_specs=pl.BlockSpec((nb_rows, D), lambda i: (i, 0)),
        out_shape=jax.ShapeDtypeStruct((n, D), jnp.float32),
    )(g3, dd, a1, bm, c1, b1, w2t, b2, gnw, gnb, m8)


def _tf_body(x_ref, w1t_ref, b1_ref, w2t_ref, b2_ref, out_ref):
    h = jnp.dot(x_ref[...], w1t_ref[...],
                preferred_element_type=jnp.float32) + b1_ref[...]
    h = jnp.maximum(h, 0.2 * h)
    out_ref[...] = jnp.dot(h, w2t_ref[...],
                           preferred_element_type=jnp.float32) + b2_ref[...]


def _tf(x, w1, b1, w2, b2):
    n = x.shape[0]
    return pl.pallas_call(
        _tf_body,
        grid=(1,),
        in_specs=[
            pl.BlockSpec((n, D), lambda i: (0, 0)),
            pl.BlockSpec((D, D), lambda i: (0, 0)),
            pl.BlockSpec((1, D), lambda i: (0, 0)),
            pl.BlockSpec((D, D), lambda i: (0, 0)),
            pl.BlockSpec((1, D), lambda i: (0, 0)),
        ],
        out_specs=pl.BlockSpec((n, D), lambda i: (0, 0)),
        out_shape=jax.ShapeDtypeStruct((n, D), jnp.float32),
    )(x, w1.T, b1[None, :], w2.T, b2[None, :])


def kernel(xyz, atom_xyz, atom_features, batch, atom_batch, params):
    p = params
    n_surf, n_atom = xyz.shape[0], atom_xyz.shape[0]
    na_pad = ((n_atom + 511) // 512) * 512
    ns_pad = ((n_surf + 511) // 512) * 512

    af_pad = jnp.pad(atom_features, ((0, na_pad - n_atom), (0, 0)))
    out = _tf(af_pad, p['tf_w1'], p['tf_b1'], p['tf_w2'], p['tf_b2'])

    idx, dist = _knn(atom_xyz, atom_xyz, atom_batch, atom_batch, K + 1)
    idx = idx[:, 1:]
    dd = jnp.pad(dist[:, 1:], ((0, na_pad - n_atom), (0, 0)))
    idx_flat = jnp.pad(idx.T.astype(jnp.int32),
                       ((0, 0), (0, na_pad - n_atom))).reshape(-1)
    for i in range(N_LAYERS):
        g = _sc_gather(out, idx_flat).reshape(K, na_pad, D)
        out = _mp_layer(out, g, dd,
                        p['aa_w1_' + str(i)], p['aa_b1_' + str(i)],
                        p['aa_w2_' + str(i)], p['aa_b2_' + str(i)],
                        p['aa_gn_w_' + str(i)], p['aa_gn_b_' + str(i)])

    idx2, dist2 = _knn(xyz, atom_xyz, batch, atom_batch, K)
    dd2 = jnp.pad(dist2, ((0, ns_pad - n_surf), (0, 0)))
    idx2_flat = jnp.pad(idx2.T.astype(jnp.int32),
                        ((0, 0), (0, ns_pad - n_surf))).reshape(-1)
    g2 = _sc_gather(out, idx2_flat).reshape(K, ns_pad, D)
    pe = _ae3(g2, dd2, p)
    return pe[:n_surf]


# speedup vs baseline: 23.0021x; 1.0126x over previous
import functools

import jax
import jax.numpy as jnp
from jax import lax
from jax.experimental import pallas as pl
from jax.experimental.pallas import tpu as pltpu
from jax.experimental.pallas import tpu_sc as plsc

D = 16
K = 16
N_LAYERS = 3
H = 2 * D + 1
MASKVAL = 1e10
DONEVAL = 3e10


_TW = 256
_IBIG = 1 << 30


def _knn_body(lohi_ref, qs_ref, ks_ref, q2_ref, k2_ref, qb_ref, kb_ref,
              idx_ref, dist_ref, *, k_sel):
    i = pl.program_id(0)
    lo = lohi_ref[0, i]
    n_tiles = lohi_ref[1, i]
    qs = qs_ref[...]
    q2 = q2_ref[...]
    qb = qb_ref[...]

    cds = tuple(jnp.full(q2.shape, MASKVAL, jnp.float32)
                for _ in range(k_sel))
    cis = tuple(jnp.full(q2.shape, p, jnp.int32) for p in range(k_sel))

    def tile_step(t, carry):
        cds, cis = carry
        off = pl.multiple_of(lo + t * _TW, 8)
        kst = ks_ref[pl.ds(off, _TW), :]
        k2t = k2_ref[pl.ds(off, _TW), :]
        kbt = kb_ref[pl.ds(off, _TW), :]
        cross = lax.dot_general(kst, qs, (((1,), (0,)), ((), ())),
                                preferred_element_type=jnp.float32)
        dt = (q2 + k2t) - 2.0 * cross
        dt = jnp.where(kbt != qb, MASKVAL, dt)
        it = lax.broadcasted_iota(jnp.int32, dt.shape, 0) + off
        ad = jnp.concatenate(list(cds) + [dt], axis=0)
        ai = jnp.concatenate(list(cis) + [it], axis=0)
        ncd, nci = [], []
        for _ in range(k_sel):
            m = jnp.min(ad, axis=0, keepdims=True)
            eq = ad == m
            ii = jnp.min(jnp.where(eq, ai, _IBIG), axis=0, keepdims=True)
            ncd.append(m)
            nci.append(ii)
            ad = jnp.where(eq, DONEVAL, ad)
        return tuple(ncd), tuple(nci)

    cds, cis = lax.fori_loop(0, n_tiles, tile_step, (cds, cis))
    for p in range(k_sel):
        dist_ref[p:p + 1, :] = cds[p]
        idx_ref[p:p + 1, :] = cis[p]


def _knn(q_xyz, k_xyz, q_batch, k_batch, k_sel, qb_rows=256):
    nq, nk = q_xyz.shape[0], k_xyz.shape[0]
    nq_pad = ((nq + qb_rows - 1) // qb_rows) * qb_rows
    nk_pad = ((nk + _TW - 1) // _TW) * _TW + _TW
    nblk = nq_pad // qb_rows
    kp = ((k_sel + 7) // 8) * 8

    qs = jnp.pad(q_xyz.T, ((0, 5), (0, nq_pad - nq)))
    ks = jnp.pad(k_xyz, ((0, nk_pad - nk), (0, 5)))
    q2 = jnp.pad(jnp.sum(q_xyz * q_xyz, axis=1), (0, nq_pad - nq))[None, :]
    k2 = jnp.pad(jnp.sum(k_xyz * k_xyz, axis=1), (0, nk_pad - nk))[:, None]
    qb = jnp.pad(q_batch.astype(jnp.int32), (0, nq_pad - nq),
                 constant_values=1 << 20)[None, :]
    kb = jnp.pad(k_batch.astype(jnp.int32), (0, nk_pad - nk),
                 constant_values=-2)[:, None]

    blo = qb[0, ::qb_rows]
    bhi = qb[0, qb_rows - 1::qb_rows]
    kstart = jnp.searchsorted(k_batch, blo).astype(jnp.int32)
    kend = jnp.searchsorted(k_batch, bhi, side='right').astype(jnp.int32)
    lo = (kstart // 8) * 8
    ntiles = (kend - lo + _TW - 1) // _TW
    lohi = jnp.stack([lo, ntiles], axis=0)

    idx, dist = pl.pallas_call(
        functools.partial(_knn_body, k_sel=k_sel),
        grid=(nblk,),
        in_specs=[
            pl.BlockSpec(memory_space=pltpu.SMEM),
            pl.BlockSpec((8, qb_rows), lambda i: (0, i)),
            pl.BlockSpec((nk_pad, 8), lambda i: (0, 0)),
            pl.BlockSpec((1, qb_rows), lambda i: (0, i)),
            pl.BlockSpec((nk_pad, 1), lambda i: (0, 0)),
            pl.BlockSpec((1, qb_rows), lambda i: (0, i)),
            pl.BlockSpec((nk_pad, 1), lambda i: (0, 0)),
        ],
        out_specs=[
            pl.BlockSpec((kp, qb_rows), lambda i: (0, i)),
            pl.BlockSpec((kp, qb_rows), lambda i: (0, i)),
        ],
        out_shape=[
            jax.ShapeDtypeStruct((kp, nq_pad), jnp.int32),
            jax.ShapeDtypeStruct((kp, nq_pad), jnp.float32),
        ],
    )(lohi, qs, ks, q2, k2, qb, kb)
    return idx[:k_sel, :nq].T, dist[:k_sel, :nq].T


def _sc_gather(table, idx):
    B = idx.shape[0]
    NW = 32
    bpw = B // NW
    CH = 128
    n_chunks = bpw // CH
    mesh = plsc.VectorSubcoreMesh(core_axis_name="c", subcore_axis_name="s")

    @functools.partial(
        pl.kernel, mesh=mesh,
        compiler_params=pltpu.CompilerParams(use_tc_tiling_on_sc=False),
        out_type=jax.ShapeDtypeStruct((B, D), jnp.float32),
        scratch_types=[
            pltpu.VMEM((bpw,), jnp.int32),
            pltpu.VMEM((bpw, D), jnp.float32),
            pltpu.SemaphoreType.DMA,
        ],
    )
    def gk(table_hbm, idx_hbm, out_hbm, idx_v, rows_v, sem):
        wid = lax.axis_index("s") * 2 + lax.axis_index("c")
        base = wid * bpw
        pltpu.sync_copy(idx_hbm.at[pl.ds(base, bpw)], idx_v)

        def fire(c, carry):
            off = c * CH
            pltpu.async_copy(table_hbm.at[idx_v.at[pl.ds(off, CH)]],
                             rows_v.at[pl.ds(off, CH)], sem)
            return carry

        lax.fori_loop(0, n_chunks, fire, 0)
        pltpu.make_async_copy(table_hbm.at[idx_v], rows_v, sem).wait()
        pltpu.sync_copy(rows_v, out_hbm.at[pl.ds(base, bpw)])

    return gk(table, idx)


def _layer_body(x_ref, g_ref, dd_ref, a1_ref, bm2_ref, s2_ref, b12_ref,
                w2t2_ref, b2_ref, gnw_ref, gnb_ref, m8_ref, out_ref,
                *, ng, gsl):
    x = x_ref[...]
    xi = jnp.dot(x, a1_ref[...],
                 preferred_element_type=jnp.float32)
    xi2 = jnp.concatenate([xi] * gsl, axis=1)
    acc = jnp.zeros_like(x)
    for g in range(ng):
        h = xi2 + jnp.dot(g_ref[g], bm2_ref[...],
                          preferred_element_type=jnp.float32)
        h = h + jnp.dot(dd_ref[:, g * gsl:(g + 1) * gsl], s2_ref[...],
                        preferred_element_type=jnp.float32)
        h = h + b12_ref[...]
        h = jnp.maximum(h, 0.2 * h)
        acc = acc + jnp.dot(h, w2t2_ref[...],
                            preferred_element_type=jnp.float32)
    msum = acc + jnp.float32(K) * b2_ref[...]
    mu = jnp.dot(msum, m8_ref[...], preferred_element_type=jnp.float32)
    cent = msum - mu
    var = jnp.dot(cent * cent, m8_ref[...],
                  preferred_element_type=jnp.float32)
    y = (cent / jnp.sqrt(var + 1e-5)) * gnw_ref[...] + gnb_ref[...]
    out_ref[...] = x + jnp.maximum(y, 0.2 * y)


def _mp_layer(x, g3, dd, pw, i, nb_rows=512):
    n = x.shape[0]
    a1, bm2, s2, b12, w2t2, b2, gnw, gnb, m8 = pw

    full = lambda s: pl.BlockSpec(s, lambda j: (0,) * len(s))
    return pl.pallas_call(
        functools.partial(_layer_body, ng=_NG, gsl=_G),
        grid=(n // nb_rows,),
        in_specs=[
            pl.BlockSpec((nb_rows, D), lambda j: (j, 0)),
            pl.BlockSpec((_NG, nb_rows, _G * D), lambda j: (0, j, 0)),
            pl.BlockSpec((nb_rows, K), lambda j: (j, 0)),
            full((D, H)), full((_G * D, _G * H)),
            full((_G, _G * H)), full((1, _G * H)),
            full((_G * H, D)), full((1, D)),
            full((1, D)), full((1, D)),
            full((D, D)),
        ],
        out_specs=pl.BlockSpec((nb_rows, D), lambda j: (j, 0)),
        out_shape=jax.ShapeDtypeStruct((n, D), jnp.float32),
    )(x, g3, dd, a1[i], bm2[i], s2[i], b12[i], w2t2[i], b2[i],
      gnw[i], gnb[i], m8)


_G = 2
_NG = K // _G


def _ae3_body(g_ref, dd_ref, a1_ref, bm2_ref, s2_ref, b12_ref,
              w2t2_ref, b2_ref, gnw_ref, gnb_ref, m8_ref, out_ref):
    x = jnp.ones_like(out_ref)
    m8 = m8_ref[...]
    for i in range(N_LAYERS):
        xi = jnp.dot(x, a1_ref[i], preferred_element_type=jnp.float32)
        xi2 = jnp.concatenate([xi] * _G, axis=1)
        acc = jnp.zeros_like(x)
        for g in range(_NG):
            h = xi2 + jnp.dot(g_ref[g], bm2_ref[i],
                              preferred_element_type=jnp.float32)
            h = h + jnp.dot(dd_ref[:, g * _G:(g + 1) * _G], s2_ref[i],
                            preferred_element_type=jnp.float32)
            h = h + b12_ref[i]
            h = jnp.maximum(h, 0.2 * h)
            acc = acc + jnp.dot(h, w2t2_ref[i],
                                preferred_element_type=jnp.float32)
        msum = acc + jnp.float32(K) * b2_ref[i]
        mu = jnp.dot(msum, m8, preferred_element_type=jnp.float32)
        cent = msum - mu
        var = jnp.dot(cent * cent, m8, preferred_element_type=jnp.float32)
        y = (cent / jnp.sqrt(var + 1e-5)) * gnw_ref[i] + gnb_ref[i]
        x = x + jnp.maximum(y, 0.2 * y)
    out_ref[...] = x


def _pack_weights(p, pre):
    a1 = jnp.stack([p[pre + '_w1_' + str(i)][:, :D].T
                    for i in range(N_LAYERS)])
    bm2, s2, b12, w2t2 = [], [], [], []
    for i in range(N_LAYERS):
        bmi = p[pre + '_w1_' + str(i)][:, D:2 * D].T
        c1i = p[pre + '_w1_' + str(i)][:, 2 * D][None, :]
        bm2.append(jax.scipy.linalg.block_diag(*([bmi] * _G)))
        s2.append(jax.scipy.linalg.block_diag(*([c1i] * _G)))
        b12.append(jnp.concatenate([p[pre + '_b1_' + str(i)]] * _G)[None, :])
        w2t2.append(jnp.concatenate([p[pre + '_w2_' + str(i)].T] * _G,
                                    axis=0))
    b2 = jnp.stack([p[pre + '_b2_' + str(i)][None, :]
                    for i in range(N_LAYERS)])
    gnw = jnp.stack([p[pre + '_gn_w_' + str(i)][None, :]
                     for i in range(N_LAYERS)])
    gnb = jnp.stack([p[pre + '_gn_b_' + str(i)][None, :]
                     for i in range(N_LAYERS)])
    m8 = jnp.zeros((D, D), jnp.float32)
    m8 = m8.at[:8, :8].set(1.0 / 8).at[8:, 8:].set(1.0 / 8)
    return (a1, jnp.stack(bm2), jnp.stack(s2), jnp.stack(b12),
            jnp.stack(w2t2), b2, gnw, gnb, m8)


def _ae3(g3, dd, p, nb_rows=512):
    n = dd.shape[0]
    a1, bm2, s2, b12, w2t2, b2, gnw, gnb, m8 = _pack_weights(p, 'ae')

    full = lambda s: pl.BlockSpec(s, lambda i: (0,) * len(s))
    return pl.pallas_call(
        _ae3_body,
        grid=(n // nb_rows,),
        in_specs=[
            pl.BlockSpec((_NG, nb_rows, _G * D), lambda i: (0, i, 0)),
            pl.BlockSpec((nb_rows, K), lambda i: (i, 0)),
            full((N_LAYERS, D, H)), full((N_LAYERS, _G * D, _G * H)),
            full((N_LAYERS, _G, _G * H)), full((N_LAYERS, 1, _G * H)),
            full((N_LAYERS, _G * H, D)), full((N_LAYERS, 1, D)),
            full((N_LAYERS, 1, D)), full((N_LAYERS, 1, D)),
            full((D, D)),
        ],
        out_specs=pl.BlockSpec((nb_rows, D), lambda i: (i, 0)),
        out_shape=jax.ShapeDtypeStruct((n, D), jnp.float32),
    )(g3, dd, a1, bm2, s2, b12, w2t2, b2, gnw, gnb, m8)


def _tf_body(x_ref, w1t_ref, b1_ref, w2t_ref, b2_ref, out_ref):
    h = jnp.dot(x_ref[...], w1t_ref[...],
                preferred_element_type=jnp.float32) + b1_ref[...]
    h = jnp.maximum(h, 0.2 * h)
    out_ref[...] = jnp.dot(h, w2t_ref[...],
                           preferred_element_type=jnp.float32) + b2_ref[...]


def _tf(x, w1, b1, w2, b2):
    n = x.shape[0]
    return pl.pallas_call(
        _tf_body,
        grid=(1,),
        in_specs=[
            pl.BlockSpec((n, D), lambda i: (0, 0)),
            pl.BlockSpec((D, D), lambda i: (0, 0)),
            pl.BlockSpec((1, D), lambda i: (0, 0)),
            pl.BlockSpec((D, D), lambda i: (0, 0)),
            pl.BlockSpec((1, D), lambda i: (0, 0)),
        ],
        out_specs=pl.BlockSpec((n, D), lambda i: (0, 0)),
        out_shape=jax.ShapeDtypeStruct((n, D), jnp.float32),
    )(x, w1.T, b1[None, :], w2.T, b2[None, :])


def kernel(xyz, atom_xyz, atom_features, batch, atom_batch, params):
    p = params
    n_surf, n_atom = xyz.shape[0], atom_xyz.shape[0]
    na_pad = ((n_atom + 511) // 512) * 512
    ns_pad = ((n_surf + 511) // 512) * 512

    af_pad = jnp.pad(atom_features, ((0, na_pad - n_atom), (0, 0)))
    out = _tf(af_pad, p['tf_w1'], p['tf_b1'], p['tf_w2'], p['tf_b2'])

    idx, dist = _knn(atom_xyz, atom_xyz, atom_batch, atom_batch, K + 1)
    dd = jnp.pad(dist[:, 1:], ((0, na_pad - n_atom), (0, 0)))
    idxn = jnp.pad(idx[:, 1:].astype(jnp.int32),
                   ((0, na_pad - n_atom), (0, 0)))
    idx_flat = idxn.reshape(na_pad, _NG, _G).transpose(1, 0, 2).reshape(-1)
    pw_aa = _pack_weights(p, 'aa')
    for i in range(N_LAYERS):
        g = _sc_gather(out, idx_flat).reshape(_NG, na_pad, _G * D)
        out = _mp_layer(out, g, dd, pw_aa, i)

    idx2, dist2 = _knn(xyz, atom_xyz, batch, atom_batch, K)
    dd2 = jnp.pad(dist2, ((0, ns_pad - n_surf), (0, 0)))
    idx2n = jnp.pad(idx2.astype(jnp.int32), ((0, ns_pad - n_surf), (0, 0)))
    idx2_flat = idx2n.reshape(ns_pad, _NG, _G).transpose(1, 0, 2).reshape(-1)
    g2 = _sc_gather(out, idx2_flat).reshape(_NG, ns_pad, _G * D)
    pe = _ae3(g2, dd2, p)
    return pe[:n_surf]
